# baseline (speedup 1.0000x reference)
import functools

import jax
import jax.numpy as jnp
from jax import lax
from jax.experimental import pallas as pl
from jax.experimental.pallas import tpu as pltpu
from jax.experimental.pallas import tpu_sc as plsc

N = 10000
E = 160000
D = 256
DH = 128
DT = 512
NSUB = 16
EPT = E // NSUB
CH = 80
NCH = EPT // CH
NPH = 5
PCH = NCH // NPH
SLAB = 624
SLAB_LAST = N - 15 * SLAB
EPW = E // 32
DCH = 40
DNCH = EPW // DCH

_mesh = plsc.VectorSubcoreMesh(core_axis_name="c", subcore_axis_name="s")


@functools.partial(
    pl.kernel,
    mesh=_mesh,
    out_type=(
        jax.ShapeDtypeStruct((N,), jnp.float32),
        jax.ShapeDtypeStruct((N,), jnp.float32),
    ),
    scratch_types=[
        pltpu.VMEM((DNCH, DCH), jnp.int32),
        pltpu.VMEM((DCH,), jnp.float32),
        pltpu.VMEM_SHARED((N,), jnp.float32),
        pltpu.SemaphoreType.DMA,
    ],
)
def _deg_kernel(edges_hbm, ones_hbm, zeros_hbm, out0_hbm, out1_hbm,
                dst_v, ones_v, acc, sem):
    c = lax.axis_index("c")
    s = lax.axis_index("s")
    wid = c * NSUB + s
    pltpu.sync_copy(edges_hbm.at[1, wid], dst_v)
    pltpu.sync_copy(ones_hbm, ones_v)

    @pl.when(s == 0)
    def _():
        pltpu.sync_copy(zeros_hbm, acc)

    plsc.subcore_barrier()

    def body(k, carry):
        pltpu.async_copy(ones_v, acc.at[dst_v.at[k]], sem, add=True)
        return carry

    lax.fori_loop(0, DNCH, body, 0)

    def drain(k, carry):
        pltpu.make_async_copy(ones_v, acc.at[dst_v.at[k]], sem).wait()
        return carry

    lax.fori_loop(0, DNCH, drain, 0)
    plsc.subcore_barrier()

    @pl.when((s == 0) & (c == 0))
    def _():
        pltpu.sync_copy(acc, out0_hbm)

    @pl.when((s == 0) & (c == 1))
    def _():
        pltpu.sync_copy(acc, out1_hbm)


@functools.partial(
    pl.kernel,
    mesh=_mesh,
    out_type=(
        jax.ShapeDtypeStruct((N, DH), jnp.float32),
        jax.ShapeDtypeStruct((N, DH), jnp.float32),
    ),
    scratch_types=[
        pltpu.VMEM((PCH, CH), jnp.int32),
        pltpu.VMEM((PCH, CH), jnp.int32),
        pltpu.VMEM((CH, DH), jnp.float32),
        pltpu.VMEM((CH, DH), jnp.float32),
        pltpu.VMEM((CH, DH), jnp.float32),
        pltpu.VMEM_SHARED((N, DH), jnp.float32),
        pltpu.SemaphoreType.DMA,
        pltpu.SemaphoreType.DMA,
        pltpu.SemaphoreType.DMA,
        pltpu.SemaphoreType.DMA,
        pltpu.SemaphoreType.DMA,
        pltpu.SemaphoreType.DMA,
    ],
)
def _scatter_kernel(ga_hbm, gb_hbm, edges_hbm, zrows_hbm,
                    outa_hbm, outb_hbm, src_v, dst_v,
                    gbuf_a, gbuf_b, gbuf_c, acc,
                    sg_a, sg_b, sg_c, ss_a, ss_b, ss_c):
    c = lax.axis_index("c")
    s = lax.axis_index("s")

    @pl.when(s < 15)
    def _():
        pltpu.sync_copy(zrows_hbm.at[pl.ds(0, SLAB)],
                        acc.at[pl.ds(s * SLAB, SLAB)])

    @pl.when(s == 15)
    def _():
        pltpu.sync_copy(zrows_hbm, acc.at[pl.ds(15 * SLAB, SLAB_LAST)])

    plsc.subcore_barrier()

    def gstart(k, buf, sem):
        @pl.when(c == 0)
        def _():
            pltpu.async_copy(ga_hbm.at[src_v.at[k]], buf, sem)

        @pl.when(c == 1)
        def _():
            pltpu.async_copy(gb_hbm.at[src_v.at[k]], buf, sem)

    def gwait(k, buf, sem):
        @pl.when(c == 0)
        def _():
            pltpu.make_async_copy(ga_hbm.at[src_v.at[k]], buf, sem).wait()

        @pl.when(c == 1)
        def _():
            pltpu.make_async_copy(gb_hbm.at[src_v.at[k]], buf, sem).wait()

    def sstart(k, buf, sem):
        pltpu.async_copy(buf, acc.at[dst_v.at[k]], sem, add=True)

    def swait(k, buf, sem):
        pltpu.make_async_copy(buf, acc.at[dst_v.at[k]], sem).wait()

    bufs = ((gbuf_a, sg_a, ss_a), (gbuf_b, sg_b, ss_b), (gbuf_c, sg_c, ss_c))
    NBUF = 3

    def phase(p, pcarry):
        pltpu.sync_copy(edges_hbm.at[0, s, p], src_v)
        pltpu.sync_copy(edges_hbm.at[1, s, p], dst_v)
        gstart(0, bufs[0][0], bufs[0][1])

        def body(k, carry):
            for r in range(NBUF):
                @pl.when(lax.rem(k, NBUF) == r)
                def _(r=r):
                    b = bufs[r]
                    bprev2 = bufs[(r + NBUF - 2) % NBUF]
                    bnext1 = bufs[(r + 1) % NBUF]
                    gwait(k, b[0], b[1])

                    @pl.when(k >= 2)
                    def _():
                        swait(k - 2, bprev2[0], bprev2[2])

                    @pl.when(k + 1 < PCH)
                    def _():
                        gstart(k + 1, bnext1[0], bnext1[1])

                    sstart(k, b[0], b[2])

            return carry

        lax.fori_loop(0, PCH, body, 0)
        for kk in (PCH - 2, PCH - 1):
            blast = bufs[kk % NBUF]
            swait(kk, blast[0], blast[2])
        return pcarry

    lax.fori_loop(0, NPH, phase, 0)
    plsc.subcore_barrier()

    @pl.when((c == 0) & (s < 15))
    def _():
        pltpu.sync_copy(acc.at[pl.ds(s * SLAB, SLAB)],
                        outa_hbm.at[pl.ds(s * SLAB, SLAB)])

    @pl.when((c == 0) & (s == 15))
    def _():
        pltpu.sync_copy(acc.at[pl.ds(15 * SLAB, SLAB_LAST)],
                        outa_hbm.at[pl.ds(15 * SLAB, SLAB_LAST)])

    @pl.when((c == 1) & (s < 15))
    def _():
        pltpu.sync_copy(acc.at[pl.ds(s * SLAB, SLAB)],
                        outb_hbm.at[pl.ds(s * SLAB, SLAB)])

    @pl.when((c == 1) & (s == 15))
    def _():
        pltpu.sync_copy(acc.at[pl.ds(15 * SLAB, SLAB_LAST)],
                        outb_hbm.at[pl.ds(15 * SLAB, SLAB_LAST)])


RB = 2000
GRID = N // RB


def _prep_body(p0_ref, p1_ref, dinv_ref):
    deg = 1.0 + p0_ref[...] + p1_ref[...]
    dinv_ref[...] = lax.rsqrt(deg)[:, None]


def _prep(p0, p1):
    return pl.pallas_call(
        _prep_body,
        out_shape=jax.ShapeDtypeStruct((N, 1), jnp.float32),
    )(p0, p1)


def _mm1_body(x_ref, te_ref, w1_ref, we_ref, be_ref, dinv_ref,
              ga_ref, gb_ref, t_ref):
    h1 = jnp.dot(x_ref[...], w1_ref[...],
                 preferred_element_type=jnp.float32)
    g1 = h1 * dinv_ref[...]
    ga_ref[...] = g1[:, :DH]
    gb_ref[...] = g1[:, DH:]
    tt = jnp.dot(te_ref[...], we_ref[...],
                 preferred_element_type=jnp.float32)
    tt = tt + be_ref[...][None, :]
    t_ref[...] = tt * jax.nn.sigmoid(tt)


def _mm1(x, t_emb, W1, We, be, dinv):
    return pl.pallas_call(
        _mm1_body,
        grid=(GRID,),
        in_specs=[
            pl.BlockSpec((RB, D), lambda i: (i, 0)),
            pl.BlockSpec((RB, DT), lambda i: (i, 0)),
            pl.BlockSpec((D, D), lambda i: (0, 0)),
            pl.BlockSpec((DT, D), lambda i: (0, 0)),
            pl.BlockSpec((D,), lambda i: (0,)),
            pl.BlockSpec((RB, 1), lambda i: (i, 0)),
        ],
        out_specs=[
            pl.BlockSpec((RB, DH), lambda i: (i, 0)),
            pl.BlockSpec((RB, DH), lambda i: (i, 0)),
            pl.BlockSpec((RB, D), lambda i: (i, 0)),
        ],
        out_shape=[
            jax.ShapeDtypeStruct((N, DH), jnp.float32),
            jax.ShapeDtypeStruct((N, DH), jnp.float32),
            jax.ShapeDtypeStruct((N, D), jnp.float32),
        ],
    )(x, t_emb, W1, We, be, dinv)


def _mm2_body(s1a_ref, s1b_ref, ga_ref, gb_ref, t_ref, b1_ref, dinv_ref,
              w2_ref, g2a_ref, g2b_ref):
    s1 = jnp.concatenate([s1a_ref[...], s1b_ref[...]], axis=1)
    g1 = jnp.concatenate([ga_ref[...], gb_ref[...]], axis=1)
    dinv = dinv_ref[...]
    h = dinv * (s1 + g1) + b1_ref[...][None, :] + t_ref[...]
    g2 = jnp.dot(h, w2_ref[...],
                 preferred_element_type=jnp.float32) * dinv
    g2a_ref[...] = g2[:, :DH]
    g2b_ref[...] = g2[:, DH:]


def _mm2(s1a, s1b, ga, gb, tval, b1, dinv, W2):
    return pl.pallas_call(
        _mm2_body,
        grid=(GRID,),
        in_specs=[
            pl.BlockSpec((RB, DH), lambda i: (i, 0)),
            pl.BlockSpec((RB, DH), lambda i: (i, 0)),
            pl.BlockSpec((RB, DH), lambda i: (i, 0)),
            pl.BlockSpec((RB, DH), lambda i: (i, 0)),
            pl.BlockSpec((RB, D), lambda i: (i, 0)),
            pl.BlockSpec((D,), lambda i: (0,)),
            pl.BlockSpec((RB, 1), lambda i: (i, 0)),
            pl.BlockSpec((D, D), lambda i: (0, 0)),
        ],
        out_specs=[
            pl.BlockSpec((RB, DH), lambda i: (i, 0)),
            pl.BlockSpec((RB, DH), lambda i: (i, 0)),
        ],
        out_shape=[
            jax.ShapeDtypeStruct((N, DH), jnp.float32),
            jax.ShapeDtypeStruct((N, DH), jnp.float32),
        ],
    )(s1a, s1b, ga, gb, tval, b1, dinv, W2)


def _final_body(x_ref, s2a_ref, s2b_ref, g2a_ref, g2b_ref, b2_ref, dinv_ref,
                out_ref):
    s2 = jnp.concatenate([s2a_ref[...], s2b_ref[...]], axis=1)
    g2 = jnp.concatenate([g2a_ref[...], g2b_ref[...]], axis=1)
    pre = dinv_ref[...] * (s2 + g2) + b2_ref[...][None, :]
    out_ref[...] = x_ref[...] + pre * jax.nn.sigmoid(pre)


def _final(x, s2a, s2b, g2a, g2b, b2, dinv):
    return pl.pallas_call(
        _final_body,
        grid=(GRID,),
        in_specs=[
            pl.BlockSpec((RB, D), lambda i: (i, 0)),
            pl.BlockSpec((RB, DH), lambda i: (i, 0)),
            pl.BlockSpec((RB, DH), lambda i: (i, 0)),
            pl.BlockSpec((RB, DH), lambda i: (i, 0)),
            pl.BlockSpec((RB, DH), lambda i: (i, 0)),
            pl.BlockSpec((D,), lambda i: (0,)),
            pl.BlockSpec((RB, 1), lambda i: (i, 0)),
        ],
        out_specs=pl.BlockSpec((RB, D), lambda i: (i, 0)),
        out_shape=jax.ShapeDtypeStruct((N, D), jnp.float32),
    )(x, s2a, s2b, g2a, g2b, b2, dinv)


def kernel(x, edge_index, t_emb, W1, b1, W2, b2, We, be):
    edges_deg = edge_index.reshape(2, 32, DNCH, DCH)
    edges_sc = edge_index.reshape(2, NSUB, NPH, PCH, CH)
    ones_d = jnp.ones((DCH,), jnp.float32)
    zeros_n = jnp.zeros((N,), jnp.float32)
    zeros_rows = jnp.zeros((SLAB_LAST, DH), jnp.float32)

    p0, p1 = _deg_kernel(edges_deg, ones_d, zeros_n)
    dinv = _prep(p0, p1)
    ga, gb, tval = _mm1(x, t_emb, W1, We, be, dinv)
    s1a, s1b = _scatter_kernel(ga, gb, edges_sc, zeros_rows)
    g2a, g2b = _mm2(s1a, s1b, ga, gb, tval, b1, dinv, W2)
    s2a, s2b = _scatter_kernel(g2a, g2b, edges_sc, zeros_rows)
    return _final(x, s2a, s2b, g2a, g2b, b2, dinv)


# speedup vs baseline: 1.3209x; 1.3209x over previous
import functools

import jax
import jax.numpy as jnp
from jax import lax
from jax.experimental import pallas as pl
from jax.experimental.pallas import tpu as pltpu
from jax.experimental.pallas import tpu_sc as plsc

N = 10000
E = 160000
D = 256
DH = 128
DT = 512
NSUB = 16
EPT = E // NSUB
CH = 80
NCH = EPT // CH
NPH = 5
PCH = NCH // NPH
SLAB = 624
SLAB_LAST = N - 15 * SLAB
EPW = E // 32
DCH = 40
DNCH = EPW // DCH

_mesh = plsc.VectorSubcoreMesh(core_axis_name="c", subcore_axis_name="s")


@functools.partial(
    pl.kernel,
    mesh=_mesh,
    out_type=(
        jax.ShapeDtypeStruct((N,), jnp.float32),
        jax.ShapeDtypeStruct((N,), jnp.float32),
    ),
    scratch_types=[
        pltpu.VMEM((DNCH, DCH), jnp.int32),
        pltpu.VMEM((DCH,), jnp.float32),
        pltpu.VMEM_SHARED((N,), jnp.float32),
        pltpu.SemaphoreType.DMA,
    ],
)
def _deg_kernel(edges_hbm, ones_hbm, zeros_hbm, out0_hbm, out1_hbm,
                dst_v, ones_v, acc, sem):
    c = lax.axis_index("c")
    s = lax.axis_index("s")
    wid = c * NSUB + s
    pltpu.sync_copy(edges_hbm.at[1, wid], dst_v)
    pltpu.sync_copy(ones_hbm, ones_v)

    @pl.when(s == 0)
    def _():
        pltpu.sync_copy(zeros_hbm, acc)

    plsc.subcore_barrier()

    def body(k, carry):
        pltpu.async_copy(ones_v, acc.at[dst_v.at[k]], sem, add=True)
        return carry

    lax.fori_loop(0, DNCH, body, 0)

    def drain(k, carry):
        pltpu.make_async_copy(ones_v, acc.at[dst_v.at[k]], sem).wait()
        return carry

    lax.fori_loop(0, DNCH, drain, 0)
    plsc.subcore_barrier()

    @pl.when((s == 0) & (c == 0))
    def _():
        pltpu.sync_copy(acc, out0_hbm)

    @pl.when((s == 0) & (c == 1))
    def _():
        pltpu.sync_copy(acc, out1_hbm)


@functools.partial(
    pl.kernel,
    mesh=_mesh,
    out_type=(
        jax.ShapeDtypeStruct((N, DH), jnp.float32),
        jax.ShapeDtypeStruct((N, DH), jnp.float32),
    ),
    scratch_types=[
        pltpu.VMEM((PCH, 2, CH // 2), jnp.int32),
        pltpu.VMEM((PCH, CH), jnp.int32),
        pltpu.VMEM((CH, DH), jnp.float32),
        pltpu.VMEM((CH, DH), jnp.float32),
        pltpu.VMEM((CH, DH), jnp.float32),
        pltpu.VMEM_SHARED((N, DH), jnp.float32),
        pltpu.SemaphoreType.DMA,
        pltpu.SemaphoreType.DMA,
        pltpu.SemaphoreType.DMA,
        pltpu.SemaphoreType.DMA,
        pltpu.SemaphoreType.DMA,
        pltpu.SemaphoreType.DMA,
    ],
)
def _scatter_kernel(ga_hbm, gb_hbm, edges_hbm, edges2_hbm, zrows_hbm,
                    outa_hbm, outb_hbm, src_v, dst_v,
                    gbuf_a, gbuf_b, gbuf_c, acc,
                    sg_a, sg_b, sg_c, ss_a, ss_b, ss_c):
    c = lax.axis_index("c")
    s = lax.axis_index("s")

    @pl.when(s < 15)
    def _():
        pltpu.sync_copy(zrows_hbm.at[pl.ds(0, SLAB)],
                        acc.at[pl.ds(s * SLAB, SLAB)])

    @pl.when(s == 15)
    def _():
        pltpu.sync_copy(zrows_hbm, acc.at[pl.ds(15 * SLAB, SLAB_LAST)])

    plsc.subcore_barrier()

    HCH = CH // 2

    def gstart(k, buf, sem):
        @pl.when(c == 0)
        def _():
            pltpu.async_copy(ga_hbm.at[src_v.at[k, 0]],
                             buf.at[pl.ds(0, HCH)], sem)
            pltpu.async_copy(ga_hbm.at[src_v.at[k, 1]],
                             buf.at[pl.ds(HCH, HCH)], sem)

        @pl.when(c == 1)
        def _():
            pltpu.async_copy(gb_hbm.at[src_v.at[k, 0]],
                             buf.at[pl.ds(0, HCH)], sem)
            pltpu.async_copy(gb_hbm.at[src_v.at[k, 1]],
                             buf.at[pl.ds(HCH, HCH)], sem)

    def gwait(k, buf, sem):
        @pl.when(c == 0)
        def _():
            pltpu.make_async_copy(ga_hbm.at[src_v.at[k, 0]],
                                  buf.at[pl.ds(0, HCH)], sem).wait()
            pltpu.make_async_copy(ga_hbm.at[src_v.at[k, 1]],
                                  buf.at[pl.ds(HCH, HCH)], sem).wait()

        @pl.when(c == 1)
        def _():
            pltpu.make_async_copy(gb_hbm.at[src_v.at[k, 0]],
                                  buf.at[pl.ds(0, HCH)], sem).wait()
            pltpu.make_async_copy(gb_hbm.at[src_v.at[k, 1]],
                                  buf.at[pl.ds(HCH, HCH)], sem).wait()

    def sstart(k, buf, sem):
        pltpu.async_copy(buf, acc.at[dst_v.at[k]], sem, add=True)

    def swait(k, buf, sem):
        pltpu.make_async_copy(buf, acc.at[dst_v.at[k]], sem).wait()

    bufs = ((gbuf_a, sg_a, ss_a), (gbuf_b, sg_b, ss_b), (gbuf_c, sg_c, ss_c))
    NBUF = 3

    def phase(p, pcarry):
        pltpu.sync_copy(edges2_hbm.at[0, s, p], src_v)
        pltpu.sync_copy(edges_hbm.at[1, s, p], dst_v)
        gstart(0, bufs[0][0], bufs[0][1])
        gstart(1, bufs[1][0], bufs[1][1])

        def body(k, carry):
            for r in range(NBUF):
                @pl.when(lax.rem(k, NBUF) == r)
                def _(r=r):
                    b = bufs[r]
                    bprev = bufs[(r + NBUF - 1) % NBUF]
                    bnext2 = bufs[(r + 2) % NBUF]
                    gwait(k, b[0], b[1])

                    @pl.when(k >= 1)
                    def _():
                        swait(k - 1, bprev[0], bprev[2])

                    @pl.when(k + 2 < PCH)
                    def _():
                        gstart(k + 2, bnext2[0], bnext2[1])

                    sstart(k, b[0], b[2])

            return carry

        lax.fori_loop(0, PCH, body, 0)
        blast = bufs[(PCH - 1) % NBUF]
        swait(PCH - 1, blast[0], blast[2])
        return pcarry

    lax.fori_loop(0, NPH, phase, 0)
    plsc.subcore_barrier()

    @pl.when((c == 0) & (s < 15))
    def _():
        pltpu.sync_copy(acc.at[pl.ds(s * SLAB, SLAB)],
                        outa_hbm.at[pl.ds(s * SLAB, SLAB)])

    @pl.when((c == 0) & (s == 15))
    def _():
        pltpu.sync_copy(acc.at[pl.ds(15 * SLAB, SLAB_LAST)],
                        outa_hbm.at[pl.ds(15 * SLAB, SLAB_LAST)])

    @pl.when((c == 1) & (s < 15))
    def _():
        pltpu.sync_copy(acc.at[pl.ds(s * SLAB, SLAB)],
                        outb_hbm.at[pl.ds(s * SLAB, SLAB)])

    @pl.when((c == 1) & (s == 15))
    def _():
        pltpu.sync_copy(acc.at[pl.ds(15 * SLAB, SLAB_LAST)],
                        outb_hbm.at[pl.ds(15 * SLAB, SLAB_LAST)])


RB = 2000
GRID = N // RB


def _prep_body(p0_ref, p1_ref, dinv_ref):
    deg = 1.0 + p0_ref[...] + p1_ref[...]
    dinv_ref[...] = lax.rsqrt(deg)[:, None]


def _prep(p0, p1):
    return pl.pallas_call(
        _prep_body,
        out_shape=jax.ShapeDtypeStruct((N, 1), jnp.float32),
    )(p0, p1)


def _mm1_body(x_ref, te_ref, w1_ref, we_ref, be_ref, dinv_ref,
              ga_ref, gb_ref, t_ref):
    h1 = jnp.dot(x_ref[...], w1_ref[...],
                 preferred_element_type=jnp.float32)
    g1 = h1 * dinv_ref[...]
    ga_ref[...] = g1[:, :DH]
    gb_ref[...] = g1[:, DH:]
    tt = jnp.dot(te_ref[...], we_ref[...],
                 preferred_element_type=jnp.float32)
    tt = tt + be_ref[...][None, :]
    t_ref[...] = tt * jax.nn.sigmoid(tt)


def _mm1(x, t_emb, W1, We, be, dinv):
    return pl.pallas_call(
        _mm1_body,
        grid=(GRID,),
        in_specs=[
            pl.BlockSpec((RB, D), lambda i: (i, 0)),
            pl.BlockSpec((RB, DT), lambda i: (i, 0)),
            pl.BlockSpec((D, D), lambda i: (0, 0)),
            pl.BlockSpec((DT, D), lambda i: (0, 0)),
            pl.BlockSpec((D,), lambda i: (0,)),
            pl.BlockSpec((RB, 1), lambda i: (i, 0)),
        ],
        out_specs=[
            pl.BlockSpec((RB, DH), lambda i: (i, 0)),
            pl.BlockSpec((RB, DH), lambda i: (i, 0)),
            pl.BlockSpec((RB, D), lambda i: (i, 0)),
        ],
        out_shape=[
            jax.ShapeDtypeStruct((N, DH), jnp.float32),
            jax.ShapeDtypeStruct((N, DH), jnp.float32),
            jax.ShapeDtypeStruct((N, D), jnp.float32),
        ],
    )(x, t_emb, W1, We, be, dinv)


def _mm2_body(s1a_ref, s1b_ref, ga_ref, gb_ref, t_ref, b1_ref, dinv_ref,
              w2_ref, g2a_ref, g2b_ref):
    s1 = jnp.concatenate([s1a_ref[...], s1b_ref[...]], axis=1)
    g1 = jnp.concatenate([ga_ref[...], gb_ref[...]], axis=1)
    dinv = dinv_ref[...]
    h = dinv * (s1 + g1) + b1_ref[...][None, :] + t_ref[...]
    g2 = jnp.dot(h, w2_ref[...],
                 preferred_element_type=jnp.float32) * dinv
    g2a_ref[...] = g2[:, :DH]
    g2b_ref[...] = g2[:, DH:]


def _mm2(s1a, s1b, ga, gb, tval, b1, dinv, W2):
    return pl.pallas_call(
        _mm2_body,
        grid=(GRID,),
        in_specs=[
            pl.BlockSpec((RB, DH), lambda i: (i, 0)),
            pl.BlockSpec((RB, DH), lambda i: (i, 0)),
            pl.BlockSpec((RB, DH), lambda i: (i, 0)),
            pl.BlockSpec((RB, DH), lambda i: (i, 0)),
            pl.BlockSpec((RB, D), lambda i: (i, 0)),
            pl.BlockSpec((D,), lambda i: (0,)),
            pl.BlockSpec((RB, 1), lambda i: (i, 0)),
            pl.BlockSpec((D, D), lambda i: (0, 0)),
        ],
        out_specs=[
            pl.BlockSpec((RB, DH), lambda i: (i, 0)),
            pl.BlockSpec((RB, DH), lambda i: (i, 0)),
        ],
        out_shape=[
            jax.ShapeDtypeStruct((N, DH), jnp.float32),
            jax.ShapeDtypeStruct((N, DH), jnp.float32),
        ],
    )(s1a, s1b, ga, gb, tval, b1, dinv, W2)


def _final_body(x_ref, s2a_ref, s2b_ref, g2a_ref, g2b_ref, b2_ref, dinv_ref,
                out_ref):
    s2 = jnp.concatenate([s2a_ref[...], s2b_ref[...]], axis=1)
    g2 = jnp.concatenate([g2a_ref[...], g2b_ref[...]], axis=1)
    pre = dinv_ref[...] * (s2 + g2) + b2_ref[...][None, :]
    out_ref[...] = x_ref[...] + pre * jax.nn.sigmoid(pre)


def _final(x, s2a, s2b, g2a, g2b, b2, dinv):
    return pl.pallas_call(
        _final_body,
        grid=(GRID,),
        in_specs=[
            pl.BlockSpec((RB, D), lambda i: (i, 0)),
            pl.BlockSpec((RB, DH), lambda i: (i, 0)),
            pl.BlockSpec((RB, DH), lambda i: (i, 0)),
            pl.BlockSpec((RB, DH), lambda i: (i, 0)),
            pl.BlockSpec((RB, DH), lambda i: (i, 0)),
            pl.BlockSpec((D,), lambda i: (0,)),
            pl.BlockSpec((RB, 1), lambda i: (i, 0)),
        ],
        out_specs=pl.BlockSpec((RB, D), lambda i: (i, 0)),
        out_shape=jax.ShapeDtypeStruct((N, D), jnp.float32),
    )(x, s2a, s2b, g2a, g2b, b2, dinv)


def kernel(x, edge_index, t_emb, W1, b1, W2, b2, We, be):
    edges_deg = edge_index.reshape(2, 32, DNCH, DCH)
    edges_sc = edge_index.reshape(2, NSUB, NPH, PCH, CH)
    edges_sc2 = edge_index.reshape(2, NSUB, NPH, PCH, 2, CH // 2)
    ones_d = jnp.ones((DCH,), jnp.float32)
    zeros_n = jnp.zeros((N,), jnp.float32)
    zeros_rows = jnp.zeros((SLAB_LAST, DH), jnp.float32)

    p0, p1 = _deg_kernel(edges_deg, ones_d, zeros_n)
    dinv = _prep(p0, p1)
    ga, gb, tval = _mm1(x, t_emb, W1, We, be, dinv)
    s1a, s1b = _scatter_kernel(ga, gb, edges_sc, edges_sc2, zeros_rows)
    g2a, g2b = _mm2(s1a, s1b, ga, gb, tval, b1, dinv, W2)
    s2a, s2b = _scatter_kernel(g2a, g2b, edges_sc, edges_sc2, zeros_rows)
    return _final(x, s2a, s2b, g2a, g2b, b2, dinv)


# speedup vs baseline: 1.3616x; 1.0308x over previous
import functools

import jax
import jax.numpy as jnp
from jax import lax
from jax.experimental import pallas as pl
from jax.experimental.pallas import tpu as pltpu
from jax.experimental.pallas import tpu_sc as plsc

N = 10000
E = 160000
D = 256
DH = 128
DT = 512
NSUB = 16
EPT = E // NSUB
CH = 80
NCH = EPT // CH
NPH = 5
PCH = NCH // NPH
SLAB = 624
SLAB_LAST = N - 15 * SLAB
EPW = E // 32
DCH = 40
DNCH = EPW // DCH

_mesh = plsc.VectorSubcoreMesh(core_axis_name="c", subcore_axis_name="s")


@functools.partial(
    pl.kernel,
    mesh=_mesh,
    out_type=(
        jax.ShapeDtypeStruct((N,), jnp.float32),
        jax.ShapeDtypeStruct((N,), jnp.float32),
    ),
    scratch_types=[
        pltpu.VMEM((DNCH, DCH), jnp.int32),
        pltpu.VMEM((DCH,), jnp.float32),
        pltpu.VMEM_SHARED((N,), jnp.float32),
        pltpu.SemaphoreType.DMA,
    ],
)
def _deg_kernel(edges_hbm, ones_hbm, zeros_hbm, out0_hbm, out1_hbm,
                dst_v, ones_v, acc, sem):
    c = lax.axis_index("c")
    s = lax.axis_index("s")
    wid = c * NSUB + s
    pltpu.sync_copy(edges_hbm.at[1, wid], dst_v)
    pltpu.sync_copy(ones_hbm, ones_v)

    @pl.when(s == 0)
    def _():
        pltpu.sync_copy(zeros_hbm, acc)

    plsc.subcore_barrier()

    def body(k, carry):
        pltpu.async_copy(ones_v, acc.at[dst_v.at[k]], sem, add=True)
        return carry

    lax.fori_loop(0, DNCH, body, 0)

    def drain(k, carry):
        pltpu.make_async_copy(ones_v, acc.at[dst_v.at[k]], sem).wait()
        return carry

    lax.fori_loop(0, DNCH, drain, 0)
    plsc.subcore_barrier()

    @pl.when((s == 0) & (c == 0))
    def _():
        pltpu.sync_copy(acc, out0_hbm)

    @pl.when((s == 0) & (c == 1))
    def _():
        pltpu.sync_copy(acc, out1_hbm)


@functools.partial(
    pl.kernel,
    mesh=_mesh,
    out_type=(
        jax.ShapeDtypeStruct((N, DH), jnp.float32),
        jax.ShapeDtypeStruct((N, DH), jnp.float32),
    ),
    scratch_types=[
        pltpu.VMEM((PCH, CH), jnp.int32),
        pltpu.VMEM((PCH, CH), jnp.int32),
        pltpu.VMEM((CH, DH), jnp.float32),
        pltpu.VMEM((CH, DH), jnp.float32),
        pltpu.VMEM((CH, DH), jnp.float32),
        pltpu.VMEM((CH, DH), jnp.float32),
        pltpu.VMEM_SHARED((N, DH), jnp.float32),
        pltpu.SemaphoreType.DMA,
        pltpu.SemaphoreType.DMA,
        pltpu.SemaphoreType.DMA,
        pltpu.SemaphoreType.DMA,
        pltpu.SemaphoreType.DMA,
        pltpu.SemaphoreType.DMA,
        pltpu.SemaphoreType.DMA,
        pltpu.SemaphoreType.DMA,
    ],
)
def _scatter_kernel(ga_hbm, gb_hbm, edges_hbm, edges2_hbm, zrows_hbm,
                    outa_hbm, outb_hbm, src_v, dst_v,
                    gbuf_a, gbuf_b, gbuf_c, gbuf_d, acc,
                    sg_a, sg_b, sg_c, sg_d, ss_a, ss_b, ss_c, ss_d):
    c = lax.axis_index("c")
    s = lax.axis_index("s")

    @pl.when(s < 15)
    def _():
        pltpu.sync_copy(zrows_hbm.at[pl.ds(0, SLAB)],
                        acc.at[pl.ds(s * SLAB, SLAB)])

    @pl.when(s == 15)
    def _():
        pltpu.sync_copy(zrows_hbm, acc.at[pl.ds(15 * SLAB, SLAB_LAST)])

    plsc.subcore_barrier()

    def gstart(k, buf, sem):
        @pl.when(c == 0)
        def _():
            pltpu.async_copy(ga_hbm.at[src_v.at[k]], buf, sem)

        @pl.when(c == 1)
        def _():
            pltpu.async_copy(gb_hbm.at[src_v.at[k]], buf, sem)

    def gwait(k, buf, sem):
        @pl.when(c == 0)
        def _():
            pltpu.make_async_copy(ga_hbm.at[src_v.at[k]], buf, sem).wait()

        @pl.when(c == 1)
        def _():
            pltpu.make_async_copy(gb_hbm.at[src_v.at[k]], buf, sem).wait()

    def sstart(k, buf, sem):
        pltpu.async_copy(buf, acc.at[dst_v.at[k]], sem, add=True)

    def swait(k, buf, sem):
        pltpu.make_async_copy(buf, acc.at[dst_v.at[k]], sem).wait()

    bufs = ((gbuf_a, sg_a, ss_a), (gbuf_b, sg_b, ss_b),
            (gbuf_c, sg_c, ss_c), (gbuf_d, sg_d, ss_d))
    NBUF = 4

    def phase(p, pcarry):
        pltpu.sync_copy(edges_hbm.at[0, s, p], src_v)
        pltpu.sync_copy(edges_hbm.at[1, s, p], dst_v)
        gstart(0, bufs[0][0], bufs[0][1])
        gstart(1, bufs[1][0], bufs[1][1])
        gstart(2, bufs[2][0], bufs[2][1])

        def body(k, carry):
            for r in range(NBUF):
                @pl.when(lax.rem(k, NBUF) == r)
                def _(r=r):
                    b = bufs[r]
                    bprev = bufs[(r + NBUF - 1) % NBUF]
                    bnext3 = bufs[(r + 3) % NBUF]
                    gwait(k, b[0], b[1])

                    @pl.when(k >= 1)
                    def _():
                        swait(k - 1, bprev[0], bprev[2])

                    @pl.when(k + 3 < PCH)
                    def _():
                        gstart(k + 3, bnext3[0], bnext3[1])

                    sstart(k, b[0], b[2])

            return carry

        lax.fori_loop(0, PCH, body, 0)
        blast = bufs[(PCH - 1) % NBUF]
        swait(PCH - 1, blast[0], blast[2])
        return pcarry

    lax.fori_loop(0, NPH, phase, 0)
    plsc.subcore_barrier()

    @pl.when((c == 0) & (s < 15))
    def _():
        pltpu.sync_copy(acc.at[pl.ds(s * SLAB, SLAB)],
                        outa_hbm.at[pl.ds(s * SLAB, SLAB)])

    @pl.when((c == 0) & (s == 15))
    def _():
        pltpu.sync_copy(acc.at[pl.ds(15 * SLAB, SLAB_LAST)],
                        outa_hbm.at[pl.ds(15 * SLAB, SLAB_LAST)])

    @pl.when((c == 1) & (s < 15))
    def _():
        pltpu.sync_copy(acc.at[pl.ds(s * SLAB, SLAB)],
                        outb_hbm.at[pl.ds(s * SLAB, SLAB)])

    @pl.when((c == 1) & (s == 15))
    def _():
        pltpu.sync_copy(acc.at[pl.ds(15 * SLAB, SLAB_LAST)],
                        outb_hbm.at[pl.ds(15 * SLAB, SLAB_LAST)])


RB = 2000
GRID = N // RB


def _prep_body(p0_ref, p1_ref, dinv_ref):
    deg = 1.0 + p0_ref[...] + p1_ref[...]
    dinv_ref[...] = lax.rsqrt(deg)[:, None]


def _prep(p0, p1):
    return pl.pallas_call(
        _prep_body,
        out_shape=jax.ShapeDtypeStruct((N, 1), jnp.float32),
    )(p0, p1)


def _mm1_body(x_ref, te_ref, w1_ref, we_ref, be_ref, dinv_ref,
              ga_ref, gb_ref, t_ref):
    h1 = jnp.dot(x_ref[...], w1_ref[...],
                 preferred_element_type=jnp.float32)
    g1 = h1 * dinv_ref[...]
    ga_ref[...] = g1[:, :DH]
    gb_ref[...] = g1[:, DH:]
    tt = jnp.dot(te_ref[...], we_ref[...],
                 preferred_element_type=jnp.float32)
    tt = tt + be_ref[...][None, :]
    t_ref[...] = tt * jax.nn.sigmoid(tt)


def _mm1(x, t_emb, W1, We, be, dinv):
    return pl.pallas_call(
        _mm1_body,
        grid=(GRID,),
        in_specs=[
            pl.BlockSpec((RB, D), lambda i: (i, 0)),
            pl.BlockSpec((RB, DT), lambda i: (i, 0)),
            pl.BlockSpec((D, D), lambda i: (0, 0)),
            pl.BlockSpec((DT, D), lambda i: (0, 0)),
            pl.BlockSpec((D,), lambda i: (0,)),
            pl.BlockSpec((RB, 1), lambda i: (i, 0)),
        ],
        out_specs=[
            pl.BlockSpec((RB, DH), lambda i: (i, 0)),
            pl.BlockSpec((RB, DH), lambda i: (i, 0)),
            pl.BlockSpec((RB, D), lambda i: (i, 0)),
        ],
        out_shape=[
            jax.ShapeDtypeStruct((N, DH), jnp.float32),
            jax.ShapeDtypeStruct((N, DH), jnp.float32),
            jax.ShapeDtypeStruct((N, D), jnp.float32),
        ],
    )(x, t_emb, W1, We, be, dinv)


def _mm2_body(s1a_ref, s1b_ref, ga_ref, gb_ref, t_ref, b1_ref, dinv_ref,
              w2_ref, g2a_ref, g2b_ref):
    s1 = jnp.concatenate([s1a_ref[...], s1b_ref[...]], axis=1)
    g1 = jnp.concatenate([ga_ref[...], gb_ref[...]], axis=1)
    dinv = dinv_ref[...]
    h = dinv * (s1 + g1) + b1_ref[...][None, :] + t_ref[...]
    g2 = jnp.dot(h, w2_ref[...],
                 preferred_element_type=jnp.float32) * dinv
    g2a_ref[...] = g2[:, :DH]
    g2b_ref[...] = g2[:, DH:]


def _mm2(s1a, s1b, ga, gb, tval, b1, dinv, W2):
    return pl.pallas_call(
        _mm2_body,
        grid=(GRID,),
        in_specs=[
            pl.BlockSpec((RB, DH), lambda i: (i, 0)),
            pl.BlockSpec((RB, DH), lambda i: (i, 0)),
            pl.BlockSpec((RB, DH), lambda i: (i, 0)),
            pl.BlockSpec((RB, DH), lambda i: (i, 0)),
            pl.BlockSpec((RB, D), lambda i: (i, 0)),
            pl.BlockSpec((D,), lambda i: (0,)),
            pl.BlockSpec((RB, 1), lambda i: (i, 0)),
            pl.BlockSpec((D, D), lambda i: (0, 0)),
        ],
        out_specs=[
            pl.BlockSpec((RB, DH), lambda i: (i, 0)),
            pl.BlockSpec((RB, DH), lambda i: (i, 0)),
        ],
        out_shape=[
            jax.ShapeDtypeStruct((N, DH), jnp.float32),
            jax.ShapeDtypeStruct((N, DH), jnp.float32),
        ],
    )(s1a, s1b, ga, gb, tval, b1, dinv, W2)


def _final_body(x_ref, s2a_ref, s2b_ref, g2a_ref, g2b_ref, b2_ref, dinv_ref,
                out_ref):
    s2 = jnp.concatenate([s2a_ref[...], s2b_ref[...]], axis=1)
    g2 = jnp.concatenate([g2a_ref[...], g2b_ref[...]], axis=1)
    pre = dinv_ref[...] * (s2 + g2) + b2_ref[...][None, :]
    out_ref[...] = x_ref[...] + pre * jax.nn.sigmoid(pre)


def _final(x, s2a, s2b, g2a, g2b, b2, dinv):
    return pl.pallas_call(
        _final_body,
        grid=(GRID,),
        in_specs=[
            pl.BlockSpec((RB, D), lambda i: (i, 0)),
            pl.BlockSpec((RB, DH), lambda i: (i, 0)),
            pl.BlockSpec((RB, DH), lambda i: (i, 0)),
            pl.BlockSpec((RB, DH), lambda i: (i, 0)),
            pl.BlockSpec((RB, DH), lambda i: (i, 0)),
            pl.BlockSpec((D,), lambda i: (0,)),
            pl.BlockSpec((RB, 1), lambda i: (i, 0)),
        ],
        out_specs=pl.BlockSpec((RB, D), lambda i: (i, 0)),
        out_shape=jax.ShapeDtypeStruct((N, D), jnp.float32),
    )(x, s2a, s2b, g2a, g2b, b2, dinv)


def kernel(x, edge_index, t_emb, W1, b1, W2, b2, We, be):
    edges_deg = edge_index.reshape(2, 32, DNCH, DCH)
    edges_sc = edge_index.reshape(2, NSUB, NPH, PCH, CH)
    edges_sc2 = edge_index.reshape(2, NSUB, NPH, PCH, 2, CH // 2)
    ones_d = jnp.ones((DCH,), jnp.float32)
    zeros_n = jnp.zeros((N,), jnp.float32)
    zeros_rows = jnp.zeros((SLAB_LAST, DH), jnp.float32)

    p0, p1 = _deg_kernel(edges_deg, ones_d, zeros_n)
    dinv = _prep(p0, p1)
    ga, gb, tval = _mm1(x, t_emb, W1, We, be, dinv)
    s1a, s1b = _scatter_kernel(ga, gb, edges_sc, edges_sc2, zeros_rows)
    g2a, g2b = _mm2(s1a, s1b, ga, gb, tval, b1, dinv, W2)
    s2a, s2b = _scatter_kernel(g2a, g2b, edges_sc, edges_sc2, zeros_rows)
    return _final(x, s2a, s2b, g2a, g2b, b2, dinv)


# speedup vs baseline: 1.3715x; 1.0072x over previous
import functools

import jax
import jax.numpy as jnp
from jax import lax
from jax.experimental import pallas as pl
from jax.experimental.pallas import tpu as pltpu
from jax.experimental.pallas import tpu_sc as plsc

N = 10000
E = 160000
D = 256
DH = 128
DT = 512
NSUB = 16
EPT = E // NSUB
CH = 80
NCH = EPT // CH
NPH = 5
PCH = NCH // NPH
SLAB = 624
SLAB_LAST = N - 15 * SLAB
EPW = E // 32
DCH = 40
DNCH = EPW // DCH

_mesh = plsc.VectorSubcoreMesh(core_axis_name="c", subcore_axis_name="s")


@functools.partial(
    pl.kernel,
    mesh=_mesh,
    out_type=(
        jax.ShapeDtypeStruct((N,), jnp.float32),
        jax.ShapeDtypeStruct((N,), jnp.float32),
    ),
    scratch_types=[
        pltpu.VMEM((DNCH, DCH), jnp.int32),
        pltpu.VMEM((DCH,), jnp.float32),
        pltpu.VMEM_SHARED((N,), jnp.float32),
        pltpu.SemaphoreType.DMA,
    ],
)
def _deg_kernel(edges_hbm, ones_hbm, zeros_hbm, out0_hbm, out1_hbm,
                dst_v, ones_v, acc, sem):
    c = lax.axis_index("c")
    s = lax.axis_index("s")
    wid = c * NSUB + s
    pltpu.sync_copy(edges_hbm.at[1, wid], dst_v)
    pltpu.sync_copy(ones_hbm, ones_v)

    @pl.when(s == 0)
    def _():
        pltpu.sync_copy(zeros_hbm, acc)

    plsc.subcore_barrier()

    def body(k, carry):
        pltpu.async_copy(ones_v, acc.at[dst_v.at[k]], sem, add=True)
        return carry

    lax.fori_loop(0, DNCH, body, 0)

    def drain(k, carry):
        pltpu.make_async_copy(ones_v, acc.at[dst_v.at[k]], sem).wait()
        return carry

    lax.fori_loop(0, DNCH, drain, 0)
    plsc.subcore_barrier()

    @pl.when((s == 0) & (c == 0))
    def _():
        pltpu.sync_copy(acc, out0_hbm)

    @pl.when((s == 0) & (c == 1))
    def _():
        pltpu.sync_copy(acc, out1_hbm)


@functools.partial(
    pl.kernel,
    mesh=_mesh,
    out_type=(
        jax.ShapeDtypeStruct((N, DH), jnp.float32),
        jax.ShapeDtypeStruct((N, DH), jnp.float32),
    ),
    scratch_types=[
        pltpu.VMEM((PCH, CH), jnp.int32),
        pltpu.VMEM((PCH, CH), jnp.int32),
        pltpu.VMEM((CH, DH), jnp.float32),
        pltpu.VMEM((CH, DH), jnp.float32),
        pltpu.VMEM((CH, DH), jnp.float32),
        pltpu.VMEM((CH, DH), jnp.float32),
        pltpu.VMEM_SHARED((N, DH), jnp.float32),
        pltpu.SemaphoreType.DMA,
        pltpu.SemaphoreType.DMA,
        pltpu.SemaphoreType.DMA,
        pltpu.SemaphoreType.DMA,
        pltpu.SemaphoreType.DMA,
        pltpu.SemaphoreType.DMA,
        pltpu.SemaphoreType.DMA,
        pltpu.SemaphoreType.DMA,
    ],
)
def _scatter_kernel(ga_hbm, gb_hbm, edges_hbm, edges2_hbm, zrows_hbm,
                    outa_hbm, outb_hbm, src_v, dst_v,
                    gbuf_a, gbuf_b, gbuf_c, gbuf_d, acc,
                    sg_a, sg_b, sg_c, sg_d, ss_a, ss_b, ss_c, ss_d):
    c = lax.axis_index("c")
    s = lax.axis_index("s")

    @pl.when(s < 15)
    def _():
        pltpu.sync_copy(zrows_hbm.at[pl.ds(0, SLAB)],
                        acc.at[pl.ds(s * SLAB, SLAB)])

    @pl.when(s == 15)
    def _():
        pltpu.sync_copy(zrows_hbm, acc.at[pl.ds(15 * SLAB, SLAB_LAST)])

    plsc.subcore_barrier()

    def gstart(k, buf, sem):
        @pl.when(c == 0)
        def _():
            pltpu.async_copy(ga_hbm.at[src_v.at[k]], buf, sem)

        @pl.when(c == 1)
        def _():
            pltpu.async_copy(gb_hbm.at[src_v.at[k]], buf, sem)

    def gwait(k, buf, sem):
        @pl.when(c == 0)
        def _():
            pltpu.make_async_copy(ga_hbm.at[src_v.at[k]], buf, sem).wait()

        @pl.when(c == 1)
        def _():
            pltpu.make_async_copy(gb_hbm.at[src_v.at[k]], buf, sem).wait()

    def sstart(k, buf, sem):
        pltpu.async_copy(buf, acc.at[dst_v.at[k]], sem, add=True)

    def swait(k, buf, sem):
        pltpu.make_async_copy(buf, acc.at[dst_v.at[k]], sem).wait()

    bufs = ((gbuf_a, sg_a, ss_a), (gbuf_b, sg_b, ss_b),
            (gbuf_c, sg_c, ss_c), (gbuf_d, sg_d, ss_d))
    NBUF = 4

    def phase(p, pcarry):
        pltpu.sync_copy(edges_hbm.at[0, s, p], src_v)
        pltpu.sync_copy(edges_hbm.at[1, s, p], dst_v)
        gstart(0, bufs[0][0], bufs[0][1])
        gstart(1, bufs[1][0], bufs[1][1])
        gstart(2, bufs[2][0], bufs[2][1])

        def body(k, carry):
            for r in range(NBUF):
                @pl.when(lax.rem(k, NBUF) == r)
                def _(r=r):
                    b = bufs[r]
                    bprev = bufs[(r + NBUF - 1) % NBUF]
                    bnext3 = bufs[(r + 3) % NBUF]
                    gwait(k, b[0], b[1])

                    @pl.when(k >= 1)
                    def _():
                        swait(k - 1, bprev[0], bprev[2])

                    @pl.when(k + 3 < PCH)
                    def _():
                        gstart(k + 3, bnext3[0], bnext3[1])

                    sstart(k, b[0], b[2])

            return carry

        lax.fori_loop(0, PCH, body, 0)
        blast = bufs[(PCH - 1) % NBUF]
        swait(PCH - 1, blast[0], blast[2])
        return pcarry

    lax.fori_loop(0, NPH, phase, 0)
    plsc.subcore_barrier()

    @pl.when((c == 0) & (s < 15))
    def _():
        pltpu.sync_copy(acc.at[pl.ds(s * SLAB, SLAB)],
                        outa_hbm.at[pl.ds(s * SLAB, SLAB)])

    @pl.when((c == 0) & (s == 15))
    def _():
        pltpu.sync_copy(acc.at[pl.ds(15 * SLAB, SLAB_LAST)],
                        outa_hbm.at[pl.ds(15 * SLAB, SLAB_LAST)])

    @pl.when((c == 1) & (s < 15))
    def _():
        pltpu.sync_copy(acc.at[pl.ds(s * SLAB, SLAB)],
                        outb_hbm.at[pl.ds(s * SLAB, SLAB)])

    @pl.when((c == 1) & (s == 15))
    def _():
        pltpu.sync_copy(acc.at[pl.ds(15 * SLAB, SLAB_LAST)],
                        outb_hbm.at[pl.ds(15 * SLAB, SLAB_LAST)])


RB = 2000
GRID = N // RB


def _prep_body(p0_ref, p1_ref, dinv_ref):
    deg = 1.0 + p0_ref[...] + p1_ref[...]
    dinv_ref[...] = lax.rsqrt(deg)[:, None]


def _prep(p0, p1):
    return pl.pallas_call(
        _prep_body,
        out_shape=jax.ShapeDtypeStruct((N, 1), jnp.float32),
    )(p0, p1)


def _mm1_body(x_ref, w1_ref, dinv_ref, ga_ref, gb_ref):
    h1 = jnp.dot(x_ref[...], w1_ref[...],
                 preferred_element_type=jnp.float32)
    g1 = h1 * dinv_ref[...]
    ga_ref[...] = g1[:, :DH]
    gb_ref[...] = g1[:, DH:]


def _mm1(x, W1, dinv):
    return pl.pallas_call(
        _mm1_body,
        grid=(GRID,),
        in_specs=[
            pl.BlockSpec((RB, D), lambda i: (i, 0)),
            pl.BlockSpec((D, D), lambda i: (0, 0)),
            pl.BlockSpec((RB, 1), lambda i: (i, 0)),
        ],
        out_specs=[
            pl.BlockSpec((RB, DH), lambda i: (i, 0)),
            pl.BlockSpec((RB, DH), lambda i: (i, 0)),
        ],
        out_shape=[
            jax.ShapeDtypeStruct((N, DH), jnp.float32),
            jax.ShapeDtypeStruct((N, DH), jnp.float32),
        ],
    )(x, W1, dinv)


def _mm2_body(s1a_ref, s1b_ref, ga_ref, gb_ref, te_ref, we_ref, be_ref,
              b1_ref, dinv_ref, w2_ref, g2a_ref, g2b_ref):
    s1 = jnp.concatenate([s1a_ref[...], s1b_ref[...]], axis=1)
    g1 = jnp.concatenate([ga_ref[...], gb_ref[...]], axis=1)
    dinv = dinv_ref[...]
    tt = jnp.dot(te_ref[...], we_ref[...],
                 preferred_element_type=jnp.float32)
    tt = tt + be_ref[...][None, :]
    t = tt * jax.nn.sigmoid(tt)
    h = dinv * (s1 + g1) + b1_ref[...][None, :] + t
    g2 = jnp.dot(h, w2_ref[...],
                 preferred_element_type=jnp.float32) * dinv
    g2a_ref[...] = g2[:, :DH]
    g2b_ref[...] = g2[:, DH:]


def _mm2(s1a, s1b, ga, gb, t_emb, We, be, b1, dinv, W2):
    return pl.pallas_call(
        _mm2_body,
        grid=(GRID,),
        in_specs=[
            pl.BlockSpec((RB, DH), lambda i: (i, 0)),
            pl.BlockSpec((RB, DH), lambda i: (i, 0)),
            pl.BlockSpec((RB, DH), lambda i: (i, 0)),
            pl.BlockSpec((RB, DH), lambda i: (i, 0)),
            pl.BlockSpec((RB, DT), lambda i: (i, 0)),
            pl.BlockSpec((DT, D), lambda i: (0, 0)),
            pl.BlockSpec((D,), lambda i: (0,)),
            pl.BlockSpec((D,), lambda i: (0,)),
            pl.BlockSpec((RB, 1), lambda i: (i, 0)),
            pl.BlockSpec((D, D), lambda i: (0, 0)),
        ],
        out_specs=[
            pl.BlockSpec((RB, DH), lambda i: (i, 0)),
            pl.BlockSpec((RB, DH), lambda i: (i, 0)),
        ],
        out_shape=[
            jax.ShapeDtypeStruct((N, DH), jnp.float32),
            jax.ShapeDtypeStruct((N, DH), jnp.float32),
        ],
    )(s1a, s1b, ga, gb, t_emb, We, be, b1, dinv, W2)


def _final_body(x_ref, s2a_ref, s2b_ref, g2a_ref, g2b_ref, b2_ref, dinv_ref,
                out_ref):
    s2 = jnp.concatenate([s2a_ref[...], s2b_ref[...]], axis=1)
    g2 = jnp.concatenate([g2a_ref[...], g2b_ref[...]], axis=1)
    pre = dinv_ref[...] * (s2 + g2) + b2_ref[...][None, :]
    out_ref[...] = x_ref[...] + pre * jax.nn.sigmoid(pre)


def _final(x, s2a, s2b, g2a, g2b, b2, dinv):
    return pl.pallas_call(
        _final_body,
        grid=(GRID,),
        in_specs=[
            pl.BlockSpec((RB, D), lambda i: (i, 0)),
            pl.BlockSpec((RB, DH), lambda i: (i, 0)),
            pl.BlockSpec((RB, DH), lambda i: (i, 0)),
            pl.BlockSpec((RB, DH), lambda i: (i, 0)),
            pl.BlockSpec((RB, DH), lambda i: (i, 0)),
            pl.BlockSpec((D,), lambda i: (0,)),
            pl.BlockSpec((RB, 1), lambda i: (i, 0)),
        ],
        out_specs=pl.BlockSpec((RB, D), lambda i: (i, 0)),
        out_shape=jax.ShapeDtypeStruct((N, D), jnp.float32),
    )(x, s2a, s2b, g2a, g2b, b2, dinv)


def kernel(x, edge_index, t_emb, W1, b1, W2, b2, We, be):
    edges_deg = edge_index.reshape(2, 32, DNCH, DCH)
    edges_sc = edge_index.reshape(2, NSUB, NPH, PCH, CH)
    edges_sc2 = edge_index.reshape(2, NSUB, NPH, PCH, 2, CH // 2)
    ones_d = jnp.ones((DCH,), jnp.float32)
    zeros_n = jnp.zeros((N,), jnp.float32)
    zeros_rows = jnp.zeros((SLAB_LAST, DH), jnp.float32)

    p0, p1 = _deg_kernel(edges_deg, ones_d, zeros_n)
    dinv = _prep(p0, p1)
    ga, gb = _mm1(x, W1, dinv)
    s1a, s1b = _scatter_kernel(ga, gb, edges_sc, edges_sc2, zeros_rows)
    g2a, g2b = _mm2(s1a, s1b, ga, gb, t_emb, We, be, b1, dinv, W2)
    s2a, s2b = _scatter_kernel(g2a, g2b, edges_sc, edges_sc2, zeros_rows)
    return _final(x, s2a, s2b, g2a, g2b, b2, dinv)


# speedup vs baseline: 1.4079x; 1.0266x over previous
import functools

import jax
import jax.numpy as jnp
from jax import lax
from jax.experimental import pallas as pl
from jax.experimental.pallas import tpu as pltpu
from jax.experimental.pallas import tpu_sc as plsc

N = 10000
E = 160000
D = 256
DH = 128
DT = 512
NSUB = 16
EPT = E // NSUB
CH = 80
NCH = EPT // CH
NPH = 5
PCH = NCH // NPH
SLAB = 624
SLAB_LAST = N - 15 * SLAB
EPW = E // 32
DCH = 40
DNCH = EPW // DCH

_mesh = plsc.VectorSubcoreMesh(core_axis_name="c", subcore_axis_name="s")


@functools.partial(
    pl.kernel,
    mesh=_mesh,
    out_type=(
        jax.ShapeDtypeStruct((N,), jnp.float32),
        jax.ShapeDtypeStruct((N,), jnp.float32),
    ),
    scratch_types=[
        pltpu.VMEM((DNCH, DCH), jnp.int32),
        pltpu.VMEM((DCH,), jnp.float32),
        pltpu.VMEM_SHARED((N,), jnp.float32),
        pltpu.SemaphoreType.DMA,
    ],
)
def _deg_kernel(edges_hbm, ones_hbm, zeros_hbm, out0_hbm, out1_hbm,
                dst_v, ones_v, acc, sem):
    c = lax.axis_index("c")
    s = lax.axis_index("s")
    wid = c * NSUB + s
    pltpu.sync_copy(edges_hbm.at[1, wid], dst_v)
    pltpu.sync_copy(ones_hbm, ones_v)

    @pl.when(s == 0)
    def _():
        pltpu.sync_copy(zeros_hbm, acc)

    plsc.subcore_barrier()

    def body(k, carry):
        pltpu.async_copy(ones_v, acc.at[dst_v.at[k]], sem, add=True)
        return carry

    lax.fori_loop(0, DNCH, body, 0)

    def drain(k, carry):
        pltpu.make_async_copy(ones_v, acc.at[dst_v.at[k]], sem).wait()
        return carry

    lax.fori_loop(0, DNCH, drain, 0)
    plsc.subcore_barrier()

    @pl.when((s == 0) & (c == 0))
    def _():
        pltpu.sync_copy(acc, out0_hbm)

    @pl.when((s == 0) & (c == 1))
    def _():
        pltpu.sync_copy(acc, out1_hbm)


@functools.partial(
    pl.kernel,
    mesh=_mesh,
    out_type=(
        jax.ShapeDtypeStruct((N, DH), jnp.float32),
        jax.ShapeDtypeStruct((N, DH), jnp.float32),
    ),
    scratch_types=[
        pltpu.VMEM((PCH, CH), jnp.int32),
        pltpu.VMEM((PCH, CH), jnp.int32),
        pltpu.VMEM((CH, DH), jnp.float32),
        pltpu.VMEM((CH, DH), jnp.float32),
        pltpu.VMEM((CH, DH), jnp.float32),
        pltpu.VMEM((CH, DH), jnp.float32),
        pltpu.VMEM_SHARED((N, DH), jnp.float32),
        pltpu.SemaphoreType.DMA,
        pltpu.SemaphoreType.DMA,
        pltpu.SemaphoreType.DMA,
        pltpu.SemaphoreType.DMA,
        pltpu.SemaphoreType.DMA,
        pltpu.SemaphoreType.DMA,
        pltpu.SemaphoreType.DMA,
        pltpu.SemaphoreType.DMA,
    ],
)
def _scatter_kernel(ga_hbm, gb_hbm, edges_hbm, edges2_hbm,
                    outa_hbm, outb_hbm, src_v, dst_v,
                    gbuf_a, gbuf_b, gbuf_c, gbuf_d, acc,
                    sg_a, sg_b, sg_c, sg_d, ss_a, ss_b, ss_c, ss_d):
    c = lax.axis_index("c")
    s = lax.axis_index("s")

    @pl.when((c == 0) & (s < 15))
    def _():
        pltpu.sync_copy(ga_hbm.at[pl.ds(s * SLAB, SLAB)],
                        acc.at[pl.ds(s * SLAB, SLAB)])

    @pl.when((c == 0) & (s == 15))
    def _():
        pltpu.sync_copy(ga_hbm.at[pl.ds(15 * SLAB, SLAB_LAST)],
                        acc.at[pl.ds(15 * SLAB, SLAB_LAST)])

    @pl.when((c == 1) & (s < 15))
    def _():
        pltpu.sync_copy(gb_hbm.at[pl.ds(s * SLAB, SLAB)],
                        acc.at[pl.ds(s * SLAB, SLAB)])

    @pl.when((c == 1) & (s == 15))
    def _():
        pltpu.sync_copy(gb_hbm.at[pl.ds(15 * SLAB, SLAB_LAST)],
                        acc.at[pl.ds(15 * SLAB, SLAB_LAST)])

    plsc.subcore_barrier()

    def gstart(k, buf, sem):
        @pl.when(c == 0)
        def _():
            pltpu.async_copy(ga_hbm.at[src_v.at[k]], buf, sem)

        @pl.when(c == 1)
        def _():
            pltpu.async_copy(gb_hbm.at[src_v.at[k]], buf, sem)

    def gwait(k, buf, sem):
        @pl.when(c == 0)
        def _():
            pltpu.make_async_copy(ga_hbm.at[src_v.at[k]], buf, sem).wait()

        @pl.when(c == 1)
        def _():
            pltpu.make_async_copy(gb_hbm.at[src_v.at[k]], buf, sem).wait()

    def sstart(k, buf, sem):
        pltpu.async_copy(buf, acc.at[dst_v.at[k]], sem, add=True)

    def swait(k, buf, sem):
        pltpu.make_async_copy(buf, acc.at[dst_v.at[k]], sem).wait()

    bufs = ((gbuf_a, sg_a, ss_a), (gbuf_b, sg_b, ss_b),
            (gbuf_c, sg_c, ss_c), (gbuf_d, sg_d, ss_d))
    NBUF = 4

    def phase(p, pcarry):
        pltpu.sync_copy(edges_hbm.at[0, s, p], src_v)
        pltpu.sync_copy(edges_hbm.at[1, s, p], dst_v)
        gstart(0, bufs[0][0], bufs[0][1])
        gstart(1, bufs[1][0], bufs[1][1])
        gstart(2, bufs[2][0], bufs[2][1])

        def body(k, carry):
            for r in range(NBUF):
                @pl.when(lax.rem(k, NBUF) == r)
                def _(r=r):
                    b = bufs[r]
                    bprev = bufs[(r + NBUF - 1) % NBUF]
                    bnext3 = bufs[(r + 3) % NBUF]
                    gwait(k, b[0], b[1])

                    @pl.when(k >= 1)
                    def _():
                        swait(k - 1, bprev[0], bprev[2])

                    @pl.when(k + 3 < PCH)
                    def _():
                        gstart(k + 3, bnext3[0], bnext3[1])

                    sstart(k, b[0], b[2])

            return carry

        lax.fori_loop(0, PCH, body, 0)
        blast = bufs[(PCH - 1) % NBUF]
        swait(PCH - 1, blast[0], blast[2])
        return pcarry

    lax.fori_loop(0, NPH, phase, 0)
    plsc.subcore_barrier()

    @pl.when((c == 0) & (s < 15))
    def _():
        pltpu.sync_copy(acc.at[pl.ds(s * SLAB, SLAB)],
                        outa_hbm.at[pl.ds(s * SLAB, SLAB)])

    @pl.when((c == 0) & (s == 15))
    def _():
        pltpu.sync_copy(acc.at[pl.ds(15 * SLAB, SLAB_LAST)],
                        outa_hbm.at[pl.ds(15 * SLAB, SLAB_LAST)])

    @pl.when((c == 1) & (s < 15))
    def _():
        pltpu.sync_copy(acc.at[pl.ds(s * SLAB, SLAB)],
                        outb_hbm.at[pl.ds(s * SLAB, SLAB)])

    @pl.when((c == 1) & (s == 15))
    def _():
        pltpu.sync_copy(acc.at[pl.ds(15 * SLAB, SLAB_LAST)],
                        outb_hbm.at[pl.ds(15 * SLAB, SLAB_LAST)])


RB = 2000
GRID = N // RB


def _prep_body(p0_ref, p1_ref, dinv_ref):
    deg = 1.0 + p0_ref[...] + p1_ref[...]
    dinv_ref[...] = lax.rsqrt(deg)[:, None]


def _prep(p0, p1):
    return pl.pallas_call(
        _prep_body,
        out_shape=jax.ShapeDtypeStruct((N, 1), jnp.float32),
    )(p0, p1)


def _mm1_body(x_ref, w1_ref, dinv_ref, ga_ref, gb_ref):
    h1 = jnp.dot(x_ref[...], w1_ref[...],
                 preferred_element_type=jnp.float32)
    g1 = h1 * dinv_ref[...]
    ga_ref[...] = g1[:, :DH]
    gb_ref[...] = g1[:, DH:]


def _mm1(x, W1, dinv):
    return pl.pallas_call(
        _mm1_body,
        grid=(GRID,),
        in_specs=[
            pl.BlockSpec((RB, D), lambda i: (i, 0)),
            pl.BlockSpec((D, D), lambda i: (0, 0)),
            pl.BlockSpec((RB, 1), lambda i: (i, 0)),
        ],
        out_specs=[
            pl.BlockSpec((RB, DH), lambda i: (i, 0)),
            pl.BlockSpec((RB, DH), lambda i: (i, 0)),
        ],
        out_shape=[
            jax.ShapeDtypeStruct((N, DH), jnp.float32),
            jax.ShapeDtypeStruct((N, DH), jnp.float32),
        ],
    )(x, W1, dinv)


def _mm2_body(s1a_ref, s1b_ref, te_ref, we_ref, be_ref,
              b1_ref, dinv_ref, w2_ref, g2a_ref, g2b_ref):
    s1 = jnp.concatenate([s1a_ref[...], s1b_ref[...]], axis=1)
    dinv = dinv_ref[...]
    tt = jnp.dot(te_ref[...], we_ref[...],
                 preferred_element_type=jnp.float32)
    tt = tt + be_ref[...][None, :]
    t = tt * jax.nn.sigmoid(tt)
    h = dinv * s1 + b1_ref[...][None, :] + t
    g2 = jnp.dot(h, w2_ref[...],
                 preferred_element_type=jnp.float32) * dinv
    g2a_ref[...] = g2[:, :DH]
    g2b_ref[...] = g2[:, DH:]


def _mm2(s1a, s1b, t_emb, We, be, b1, dinv, W2):
    return pl.pallas_call(
        _mm2_body,
        grid=(GRID,),
        in_specs=[
            pl.BlockSpec((RB, DH), lambda i: (i, 0)),
            pl.BlockSpec((RB, DH), lambda i: (i, 0)),
            pl.BlockSpec((RB, DT), lambda i: (i, 0)),
            pl.BlockSpec((DT, D), lambda i: (0, 0)),
            pl.BlockSpec((D,), lambda i: (0,)),
            pl.BlockSpec((D,), lambda i: (0,)),
            pl.BlockSpec((RB, 1), lambda i: (i, 0)),
            pl.BlockSpec((D, D), lambda i: (0, 0)),
        ],
        out_specs=[
            pl.BlockSpec((RB, DH), lambda i: (i, 0)),
            pl.BlockSpec((RB, DH), lambda i: (i, 0)),
        ],
        out_shape=[
            jax.ShapeDtypeStruct((N, DH), jnp.float32),
            jax.ShapeDtypeStruct((N, DH), jnp.float32),
        ],
    )(s1a, s1b, t_emb, We, be, b1, dinv, W2)


def _final_body(x_ref, s2a_ref, s2b_ref, b2_ref, dinv_ref, out_ref):
    s2 = jnp.concatenate([s2a_ref[...], s2b_ref[...]], axis=1)
    pre = dinv_ref[...] * s2 + b2_ref[...][None, :]
    out_ref[...] = x_ref[...] + pre * jax.nn.sigmoid(pre)


def _final(x, s2a, s2b, b2, dinv):
    return pl.pallas_call(
        _final_body,
        grid=(GRID,),
        in_specs=[
            pl.BlockSpec((RB, D), lambda i: (i, 0)),
            pl.BlockSpec((RB, DH), lambda i: (i, 0)),
            pl.BlockSpec((RB, DH), lambda i: (i, 0)),
            pl.BlockSpec((D,), lambda i: (0,)),
            pl.BlockSpec((RB, 1), lambda i: (i, 0)),
        ],
        out_specs=pl.BlockSpec((RB, D), lambda i: (i, 0)),
        out_shape=jax.ShapeDtypeStruct((N, D), jnp.float32),
    )(x, s2a, s2b, b2, dinv)


def kernel(x, edge_index, t_emb, W1, b1, W2, b2, We, be):
    edges_deg = edge_index.reshape(2, 32, DNCH, DCH)
    edges_sc = edge_index.reshape(2, NSUB, NPH, PCH, CH)
    edges_sc2 = edge_index.reshape(2, NSUB, NPH, PCH, 2, CH // 2)
    ones_d = jnp.ones((DCH,), jnp.float32)
    zeros_n = jnp.zeros((N,), jnp.float32)

    p0, p1 = _deg_kernel(edges_deg, ones_d, zeros_n)
    dinv = _prep(p0, p1)
    ga, gb = _mm1(x, W1, dinv)
    s1a, s1b = _scatter_kernel(ga, gb, edges_sc, edges_sc2)
    g2a, g2b = _mm2(s1a, s1b, t_emb, We, be, b1, dinv, W2)
    s2a, s2b = _scatter_kernel(g2a, g2b, edges_sc, edges_sc2)
    return _final(x, s2a, s2b, b2, dinv)


# speedup vs baseline: 1.4272x; 1.0137x over previous
import functools

import jax
import jax.numpy as jnp
from jax import lax
from jax.experimental import pallas as pl
from jax.experimental.pallas import tpu as pltpu
from jax.experimental.pallas import tpu_sc as plsc

N = 10000
E = 160000
D = 256
DH = 128
DT = 512
NSUB = 16
EPT = E // NSUB
CH = 80
NCH = EPT // CH
NPH = 5
PCH = NCH // NPH
SLAB = 624
SLAB_LAST = N - 15 * SLAB
EPW = E // 32
DCH = 40
DNCH = EPW // DCH

_mesh = plsc.VectorSubcoreMesh(core_axis_name="c", subcore_axis_name="s")


@functools.partial(
    pl.kernel,
    mesh=_mesh,
    out_type=(
        jax.ShapeDtypeStruct((N,), jnp.float32),
        jax.ShapeDtypeStruct((N,), jnp.float32),
    ),
    scratch_types=[
        pltpu.VMEM((DNCH, DCH), jnp.int32),
        pltpu.VMEM((DCH,), jnp.float32),
        pltpu.VMEM_SHARED((N,), jnp.float32),
        pltpu.SemaphoreType.DMA,
    ],
)
def _deg_kernel(edges_hbm, ones_hbm, zeros_hbm, out0_hbm, out1_hbm,
                dst_v, ones_v, acc, sem):
    c = lax.axis_index("c")
    s = lax.axis_index("s")
    wid = c * NSUB + s
    pltpu.sync_copy(edges_hbm.at[1, wid], dst_v)
    pltpu.sync_copy(ones_hbm, ones_v)

    @pl.when(s == 0)
    def _():
        pltpu.sync_copy(zeros_hbm, acc)

    plsc.subcore_barrier()

    def body(k, carry):
        pltpu.async_copy(ones_v, acc.at[dst_v.at[k]], sem, add=True)
        return carry

    lax.fori_loop(0, DNCH, body, 0)

    def drain(k, carry):
        pltpu.make_async_copy(ones_v, acc.at[dst_v.at[k]], sem).wait()
        return carry

    lax.fori_loop(0, DNCH, drain, 0)
    plsc.subcore_barrier()

    @pl.when((s == 0) & (c == 0))
    def _():
        pltpu.sync_copy(acc, out0_hbm)

    @pl.when((s == 0) & (c == 1))
    def _():
        pltpu.sync_copy(acc, out1_hbm)


@functools.partial(
    pl.kernel,
    mesh=_mesh,
    out_type=(
        jax.ShapeDtypeStruct((N, DH), jnp.float32),
        jax.ShapeDtypeStruct((N, DH), jnp.float32),
    ),
    scratch_types=[
        pltpu.VMEM((PCH, CH), jnp.int32),
        pltpu.VMEM((PCH, CH), jnp.int32),
        pltpu.VMEM((CH, DH), jnp.float32),
        pltpu.VMEM((CH, DH), jnp.float32),
        pltpu.VMEM((CH, DH), jnp.float32),
        pltpu.VMEM((CH, DH), jnp.float32),
        pltpu.VMEM_SHARED((N, DH), jnp.float32),
        pltpu.SemaphoreType.DMA,
        pltpu.SemaphoreType.DMA,
        pltpu.SemaphoreType.DMA,
        pltpu.SemaphoreType.DMA,
        pltpu.SemaphoreType.DMA,
        pltpu.SemaphoreType.DMA,
        pltpu.SemaphoreType.DMA,
        pltpu.SemaphoreType.DMA,
    ],
)
def _scatter_kernel(ga_hbm, gb_hbm, edges_hbm, edges2_hbm,
                    outa_hbm, outb_hbm, src_v, dst_v,
                    gbuf_a, gbuf_b, gbuf_c, gbuf_d, acc,
                    sg_a, sg_b, sg_c, sg_d, ss_a, ss_b, ss_c, ss_d):
    c = lax.axis_index("c")
    s = lax.axis_index("s")

    @pl.when((c == 0) & (s < 15))
    def _():
        pltpu.sync_copy(ga_hbm.at[pl.ds(s * SLAB, SLAB)],
                        acc.at[pl.ds(s * SLAB, SLAB)])

    @pl.when((c == 0) & (s == 15))
    def _():
        pltpu.sync_copy(ga_hbm.at[pl.ds(15 * SLAB, SLAB_LAST)],
                        acc.at[pl.ds(15 * SLAB, SLAB_LAST)])

    @pl.when((c == 1) & (s < 15))
    def _():
        pltpu.sync_copy(gb_hbm.at[pl.ds(s * SLAB, SLAB)],
                        acc.at[pl.ds(s * SLAB, SLAB)])

    @pl.when((c == 1) & (s == 15))
    def _():
        pltpu.sync_copy(gb_hbm.at[pl.ds(15 * SLAB, SLAB_LAST)],
                        acc.at[pl.ds(15 * SLAB, SLAB_LAST)])

    plsc.subcore_barrier()

    def gstart(k, buf, sem):
        @pl.when(c == 0)
        def _():
            pltpu.async_copy(ga_hbm.at[src_v.at[k]], buf, sem)

        @pl.when(c == 1)
        def _():
            pltpu.async_copy(gb_hbm.at[src_v.at[k]], buf, sem)

    def gwait(k, buf, sem):
        @pl.when(c == 0)
        def _():
            pltpu.make_async_copy(ga_hbm.at[src_v.at[k]], buf, sem).wait()

        @pl.when(c == 1)
        def _():
            pltpu.make_async_copy(gb_hbm.at[src_v.at[k]], buf, sem).wait()

    def sstart(k, buf, sem):
        pltpu.async_copy(buf, acc.at[dst_v.at[k]], sem, add=True)

    def swait(k, buf, sem):
        pltpu.make_async_copy(buf, acc.at[dst_v.at[k]], sem).wait()

    bufs = ((gbuf_a, sg_a, ss_a), (gbuf_b, sg_b, ss_b),
            (gbuf_c, sg_c, ss_c), (gbuf_d, sg_d, ss_d))
    NBUF = 4

    def phase(p, pcarry):
        pltpu.sync_copy(edges_hbm.at[0, s, p], src_v)
        pltpu.sync_copy(edges_hbm.at[1, s, p], dst_v)
        gstart(0, bufs[0][0], bufs[0][1])
        gstart(1, bufs[1][0], bufs[1][1])
        gstart(2, bufs[2][0], bufs[2][1])

        def body(k, carry):
            for r in range(NBUF):
                @pl.when(lax.rem(k, NBUF) == r)
                def _(r=r):
                    b = bufs[r]
                    bprev = bufs[(r + NBUF - 1) % NBUF]
                    bnext3 = bufs[(r + 3) % NBUF]
                    gwait(k, b[0], b[1])

                    @pl.when(k >= 1)
                    def _():
                        swait(k - 1, bprev[0], bprev[2])

                    @pl.when(k + 3 < PCH)
                    def _():
                        gstart(k + 3, bnext3[0], bnext3[1])

                    sstart(k, b[0], b[2])

            return carry

        lax.fori_loop(0, PCH, body, 0)
        blast = bufs[(PCH - 1) % NBUF]
        swait(PCH - 1, blast[0], blast[2])
        return pcarry

    lax.fori_loop(0, NPH, phase, 0)
    plsc.subcore_barrier()

    @pl.when((c == 0) & (s < 15))
    def _():
        pltpu.sync_copy(acc.at[pl.ds(s * SLAB, SLAB)],
                        outa_hbm.at[pl.ds(s * SLAB, SLAB)])

    @pl.when((c == 0) & (s == 15))
    def _():
        pltpu.sync_copy(acc.at[pl.ds(15 * SLAB, SLAB_LAST)],
                        outa_hbm.at[pl.ds(15 * SLAB, SLAB_LAST)])

    @pl.when((c == 1) & (s < 15))
    def _():
        pltpu.sync_copy(acc.at[pl.ds(s * SLAB, SLAB)],
                        outb_hbm.at[pl.ds(s * SLAB, SLAB)])

    @pl.when((c == 1) & (s == 15))
    def _():
        pltpu.sync_copy(acc.at[pl.ds(15 * SLAB, SLAB_LAST)],
                        outb_hbm.at[pl.ds(15 * SLAB, SLAB_LAST)])


RB = 2000
GRID = N // RB


def _mm1_body(p0_ref, p1_ref, x_ref, w1_ref, ga_ref, gb_ref, dinv_ref,
              dinv_scr):
    i = pl.program_id(0)

    @pl.when(i == 0)
    def _():
        deg = 1.0 + p0_ref[...] + p1_ref[...]
        dinv_scr[...] = lax.rsqrt(deg)[:, None]

    @pl.when(i > 0)
    def _():
        r = i - 1
        dinv = dinv_scr[pl.ds(r * RB, RB), :]
        h1 = jnp.dot(x_ref[...], w1_ref[...],
                     preferred_element_type=jnp.float32)
        g1 = h1 * dinv
        ga_ref[...] = g1[:, :DH]
        gb_ref[...] = g1[:, DH:]
        dinv_ref[...] = dinv


def _mm1(p0, p1, x, W1):
    return pl.pallas_call(
        _mm1_body,
        grid=(GRID + 1,),
        in_specs=[
            pl.BlockSpec((N,), lambda i: (0,)),
            pl.BlockSpec((N,), lambda i: (0,)),
            pl.BlockSpec((RB, D), lambda i: (jnp.maximum(i - 1, 0), 0)),
            pl.BlockSpec((D, D), lambda i: (0, 0)),
        ],
        out_specs=[
            pl.BlockSpec((RB, DH), lambda i: (jnp.maximum(i - 1, 0), 0)),
            pl.BlockSpec((RB, DH), lambda i: (jnp.maximum(i - 1, 0), 0)),
            pl.BlockSpec((RB, 1), lambda i: (jnp.maximum(i - 1, 0), 0)),
        ],
        out_shape=[
            jax.ShapeDtypeStruct((N, DH), jnp.float32),
            jax.ShapeDtypeStruct((N, DH), jnp.float32),
            jax.ShapeDtypeStruct((N, 1), jnp.float32),
        ],
        scratch_shapes=[pltpu.VMEM((N, 1), jnp.float32)],
    )(p0, p1, x, W1)


def _mm2_body(s1a_ref, s1b_ref, te_ref, we_ref, be_ref,
              b1_ref, dinv_ref, w2_ref, g2a_ref, g2b_ref):
    s1 = jnp.concatenate([s1a_ref[...], s1b_ref[...]], axis=1)
    dinv = dinv_ref[...]
    tt = jnp.dot(te_ref[...], we_ref[...],
                 preferred_element_type=jnp.float32)
    tt = tt + be_ref[...][None, :]
    t = tt * jax.nn.sigmoid(tt)
    h = dinv * s1 + b1_ref[...][None, :] + t
    g2 = jnp.dot(h, w2_ref[...],
                 preferred_element_type=jnp.float32) * dinv
    g2a_ref[...] = g2[:, :DH]
    g2b_ref[...] = g2[:, DH:]


def _mm2(s1a, s1b, t_emb, We, be, b1, dinv, W2):
    return pl.pallas_call(
        _mm2_body,
        grid=(GRID,),
        in_specs=[
            pl.BlockSpec((RB, DH), lambda i: (i, 0)),
            pl.BlockSpec((RB, DH), lambda i: (i, 0)),
            pl.BlockSpec((RB, DT), lambda i: (i, 0)),
            pl.BlockSpec((DT, D), lambda i: (0, 0)),
            pl.BlockSpec((D,), lambda i: (0,)),
            pl.BlockSpec((D,), lambda i: (0,)),
            pl.BlockSpec((RB, 1), lambda i: (i, 0)),
            pl.BlockSpec((D, D), lambda i: (0, 0)),
        ],
        out_specs=[
            pl.BlockSpec((RB, DH), lambda i: (i, 0)),
            pl.BlockSpec((RB, DH), lambda i: (i, 0)),
        ],
        out_shape=[
            jax.ShapeDtypeStruct((N, DH), jnp.float32),
            jax.ShapeDtypeStruct((N, DH), jnp.float32),
        ],
    )(s1a, s1b, t_emb, We, be, b1, dinv, W2)


def _final_body(x_ref, s2a_ref, s2b_ref, b2_ref, dinv_ref, out_ref):
    s2 = jnp.concatenate([s2a_ref[...], s2b_ref[...]], axis=1)
    pre = dinv_ref[...] * s2 + b2_ref[...][None, :]
    out_ref[...] = x_ref[...] + pre * jax.nn.sigmoid(pre)


def _final(x, s2a, s2b, b2, dinv):
    return pl.pallas_call(
        _final_body,
        grid=(GRID,),
        in_specs=[
            pl.BlockSpec((RB, D), lambda i: (i, 0)),
            pl.BlockSpec((RB, DH), lambda i: (i, 0)),
            pl.BlockSpec((RB, DH), lambda i: (i, 0)),
            pl.BlockSpec((D,), lambda i: (0,)),
            pl.BlockSpec((RB, 1), lambda i: (i, 0)),
        ],
        out_specs=pl.BlockSpec((RB, D), lambda i: (i, 0)),
        out_shape=jax.ShapeDtypeStruct((N, D), jnp.float32),
    )(x, s2a, s2b, b2, dinv)


def kernel(x, edge_index, t_emb, W1, b1, W2, b2, We, be):
    edges_deg = edge_index.reshape(2, 32, DNCH, DCH)
    edges_sc = edge_index.reshape(2, NSUB, NPH, PCH, CH)
    edges_sc2 = edge_index.reshape(2, NSUB, NPH, PCH, 2, CH // 2)
    ones_d = jnp.ones((DCH,), jnp.float32)
    zeros_n = jnp.zeros((N,), jnp.float32)

    p0, p1 = _deg_kernel(edges_deg, ones_d, zeros_n)
    ga, gb, dinv = _mm1(p0, p1, x, W1)
    s1a, s1b = _scatter_kernel(ga, gb, edges_sc, edges_sc2)
    g2a, g2b = _mm2(s1a, s1b, t_emb, We, be, b1, dinv, W2)
    s2a, s2b = _scatter_kernel(g2a, g2b, edges_sc, edges_sc2)
    return _final(x, s2a, s2b, b2, dinv)


# speedup vs baseline: 1.4471x; 1.0139x over previous
import functools

import jax
import jax.numpy as jnp
from jax import lax
from jax.experimental import pallas as pl
from jax.experimental.pallas import tpu as pltpu
from jax.experimental.pallas import tpu_sc as plsc

N = 10000
E = 160000
D = 256
DH = 128
DT = 512
NSUB = 16
EPT = E // NSUB
CH = 80
NCH = EPT // CH
NPH = 5
PCH = NCH // NPH
SLAB = 624
SLAB_LAST = N - 15 * SLAB
EPW = E // 32
DCH = 40
DNCH = EPW // DCH

_mesh = plsc.VectorSubcoreMesh(core_axis_name="c", subcore_axis_name="s")


@functools.partial(
    pl.kernel,
    mesh=_mesh,
    out_type=(
        jax.ShapeDtypeStruct((N,), jnp.float32),
        jax.ShapeDtypeStruct((N,), jnp.float32),
    ),
    scratch_types=[
        pltpu.VMEM((DNCH, DCH), jnp.int32),
        pltpu.VMEM((DCH,), jnp.float32),
        pltpu.VMEM_SHARED((N,), jnp.float32),
        pltpu.SemaphoreType.DMA,
    ],
)
def _deg_kernel(edges_hbm, ones_hbm, zeros_hbm, out0_hbm, out1_hbm,
                dst_v, ones_v, acc, sem):
    c = lax.axis_index("c")
    s = lax.axis_index("s")
    wid = c * NSUB + s
    pltpu.sync_copy(edges_hbm.at[1, wid], dst_v)
    pltpu.sync_copy(ones_hbm, ones_v)

    @pl.when(s == 0)
    def _():
        pltpu.sync_copy(zeros_hbm, acc)

    plsc.subcore_barrier()

    def body(k, carry):
        pltpu.async_copy(ones_v, acc.at[dst_v.at[k]], sem, add=True)
        return carry

    lax.fori_loop(0, DNCH, body, 0)

    def drain(k, carry):
        pltpu.make_async_copy(ones_v, acc.at[dst_v.at[k]], sem).wait()
        return carry

    lax.fori_loop(0, DNCH, drain, 0)
    plsc.subcore_barrier()

    @pl.when((s == 0) & (c == 0))
    def _():
        pltpu.sync_copy(acc, out0_hbm)

    @pl.when((s == 0) & (c == 1))
    def _():
        pltpu.sync_copy(acc, out1_hbm)


@functools.partial(
    pl.kernel,
    mesh=_mesh,
    out_type=(
        jax.ShapeDtypeStruct((N, DH), jnp.float32),
        jax.ShapeDtypeStruct((N, DH), jnp.float32),
    ),
    scratch_types=[
        pltpu.VMEM((PCH, CH), jnp.int32),
        pltpu.VMEM((PCH, CH), jnp.int32),
        pltpu.VMEM((CH, DH), jnp.float32),
        pltpu.VMEM((CH, DH), jnp.float32),
        pltpu.VMEM((CH, DH), jnp.float32),
        pltpu.VMEM((CH, DH), jnp.float32),
        pltpu.VMEM_SHARED((N, DH), jnp.float32),
        pltpu.SemaphoreType.DMA,
        pltpu.SemaphoreType.DMA,
        pltpu.SemaphoreType.DMA,
        pltpu.SemaphoreType.DMA,
        pltpu.SemaphoreType.DMA,
        pltpu.SemaphoreType.DMA,
        pltpu.SemaphoreType.DMA,
        pltpu.SemaphoreType.DMA,
    ],
)
def _scatter_kernel(ga_hbm, gb_hbm, edges_hbm, edges2_hbm,
                    outa_hbm, outb_hbm, src_v, dst_v,
                    gbuf_a, gbuf_b, gbuf_c, gbuf_d, acc,
                    sg_a, sg_b, sg_c, sg_d, ss_a, ss_b, ss_c, ss_d):
    c = lax.axis_index("c")
    s = lax.axis_index("s")

    def init_selfloop():
        @pl.when((c == 0) & (s < 15))
        def _():
            pltpu.sync_copy(ga_hbm.at[pl.ds(s * SLAB, SLAB)],
                            acc.at[pl.ds(s * SLAB, SLAB)])

        @pl.when((c == 0) & (s == 15))
        def _():
            pltpu.sync_copy(ga_hbm.at[pl.ds(15 * SLAB, SLAB_LAST)],
                            acc.at[pl.ds(15 * SLAB, SLAB_LAST)])

        @pl.when((c == 1) & (s < 15))
        def _():
            pltpu.sync_copy(gb_hbm.at[pl.ds(s * SLAB, SLAB)],
                            acc.at[pl.ds(s * SLAB, SLAB)])

        @pl.when((c == 1) & (s == 15))
        def _():
            pltpu.sync_copy(gb_hbm.at[pl.ds(15 * SLAB, SLAB_LAST)],
                            acc.at[pl.ds(15 * SLAB, SLAB_LAST)])

    def gstart(k, buf, sem):
        @pl.when(c == 0)
        def _():
            pltpu.async_copy(ga_hbm.at[src_v.at[k]], buf, sem)

        @pl.when(c == 1)
        def _():
            pltpu.async_copy(gb_hbm.at[src_v.at[k]], buf, sem)

    def gwait(k, buf, sem):
        @pl.when(c == 0)
        def _():
            pltpu.make_async_copy(ga_hbm.at[src_v.at[k]], buf, sem).wait()

        @pl.when(c == 1)
        def _():
            pltpu.make_async_copy(gb_hbm.at[src_v.at[k]], buf, sem).wait()

    def sstart(k, buf, sem):
        pltpu.async_copy(buf, acc.at[dst_v.at[k]], sem, add=True)

    def swait(k, buf, sem):
        pltpu.make_async_copy(buf, acc.at[dst_v.at[k]], sem).wait()

    bufs = ((gbuf_a, sg_a, ss_a), (gbuf_b, sg_b, ss_b),
            (gbuf_c, sg_c, ss_c), (gbuf_d, sg_d, ss_d))
    NBUF = 4

    def phase(p, pcarry):
        pltpu.sync_copy(edges_hbm.at[0, s, p], src_v)
        pltpu.sync_copy(edges_hbm.at[1, s, p], dst_v)
        gstart(0, bufs[0][0], bufs[0][1])
        gstart(1, bufs[1][0], bufs[1][1])
        gstart(2, bufs[2][0], bufs[2][1])

        @pl.when(p == 0)
        def _():
            init_selfloop()
            plsc.subcore_barrier()

        def body(k, carry):
            for r in range(NBUF):
                @pl.when(lax.rem(k, NBUF) == r)
                def _(r=r):
                    b = bufs[r]
                    bprev = bufs[(r + NBUF - 1) % NBUF]
                    bnext3 = bufs[(r + 3) % NBUF]
                    gwait(k, b[0], b[1])

                    @pl.when(k >= 1)
                    def _():
                        swait(k - 1, bprev[0], bprev[2])

                    @pl.when(k + 3 < PCH)
                    def _():
                        gstart(k + 3, bnext3[0], bnext3[1])

                    sstart(k, b[0], b[2])

            return carry

        lax.fori_loop(0, PCH, body, 0)
        blast = bufs[(PCH - 1) % NBUF]
        swait(PCH - 1, blast[0], blast[2])
        return pcarry

    lax.fori_loop(0, NPH, phase, 0)
    plsc.subcore_barrier()

    @pl.when((c == 0) & (s < 15))
    def _():
        pltpu.sync_copy(acc.at[pl.ds(s * SLAB, SLAB)],
                        outa_hbm.at[pl.ds(s * SLAB, SLAB)])

    @pl.when((c == 0) & (s == 15))
    def _():
        pltpu.sync_copy(acc.at[pl.ds(15 * SLAB, SLAB_LAST)],
                        outa_hbm.at[pl.ds(15 * SLAB, SLAB_LAST)])

    @pl.when((c == 1) & (s < 15))
    def _():
        pltpu.sync_copy(acc.at[pl.ds(s * SLAB, SLAB)],
                        outb_hbm.at[pl.ds(s * SLAB, SLAB)])

    @pl.when((c == 1) & (s == 15))
    def _():
        pltpu.sync_copy(acc.at[pl.ds(15 * SLAB, SLAB_LAST)],
                        outb_hbm.at[pl.ds(15 * SLAB, SLAB_LAST)])


RB = 2000
GRID = N // RB


def _mm1_body(p0_ref, p1_ref, x_ref, w1_ref, ga_ref, gb_ref, dinv_ref,
              dinv_scr):
    i = pl.program_id(0)

    @pl.when(i == 0)
    def _():
        deg = 1.0 + p0_ref[...] + p1_ref[...]
        dinv_scr[...] = lax.rsqrt(deg)[:, None]

    @pl.when(i > 0)
    def _():
        r = i - 1
        dinv = dinv_scr[pl.ds(r * RB, RB), :]
        h1 = jnp.dot(x_ref[...], w1_ref[...],
                     preferred_element_type=jnp.float32)
        g1 = h1 * dinv
        ga_ref[...] = g1[:, :DH]
        gb_ref[...] = g1[:, DH:]
        dinv_ref[...] = dinv


def _mm1(p0, p1, x, W1):
    return pl.pallas_call(
        _mm1_body,
        grid=(GRID + 1,),
        in_specs=[
            pl.BlockSpec((N,), lambda i: (0,)),
            pl.BlockSpec((N,), lambda i: (0,)),
            pl.BlockSpec((RB, D), lambda i: (jnp.maximum(i - 1, 0), 0)),
            pl.BlockSpec((D, D), lambda i: (0, 0)),
        ],
        out_specs=[
            pl.BlockSpec((RB, DH), lambda i: (jnp.maximum(i - 1, 0), 0)),
            pl.BlockSpec((RB, DH), lambda i: (jnp.maximum(i - 1, 0), 0)),
            pl.BlockSpec((RB, 1), lambda i: (jnp.maximum(i - 1, 0), 0)),
        ],
        out_shape=[
            jax.ShapeDtypeStruct((N, DH), jnp.float32),
            jax.ShapeDtypeStruct((N, DH), jnp.float32),
            jax.ShapeDtypeStruct((N, 1), jnp.float32),
        ],
        scratch_shapes=[pltpu.VMEM((N, 1), jnp.float32)],
    )(p0, p1, x, W1)


def _mm2_body(s1a_ref, s1b_ref, te_ref, we_ref, be_ref,
              b1_ref, dinv_ref, w2_ref, g2a_ref, g2b_ref):
    s1 = jnp.concatenate([s1a_ref[...], s1b_ref[...]], axis=1)
    dinv = dinv_ref[...]
    tt = jnp.dot(te_ref[...], we_ref[...],
                 preferred_element_type=jnp.float32)
    tt = tt + be_ref[...][None, :]
    t = tt * jax.nn.sigmoid(tt)
    h = dinv * s1 + b1_ref[...][None, :] + t
    g2 = jnp.dot(h, w2_ref[...],
                 preferred_element_type=jnp.float32) * dinv
    g2a_ref[...] = g2[:, :DH]
    g2b_ref[...] = g2[:, DH:]


def _mm2(s1a, s1b, t_emb, We, be, b1, dinv, W2):
    return pl.pallas_call(
        _mm2_body,
        grid=(GRID,),
        in_specs=[
            pl.BlockSpec((RB, DH), lambda i: (i, 0)),
            pl.BlockSpec((RB, DH), lambda i: (i, 0)),
            pl.BlockSpec((RB, DT), lambda i: (i, 0)),
            pl.BlockSpec((DT, D), lambda i: (0, 0)),
            pl.BlockSpec((D,), lambda i: (0,)),
            pl.BlockSpec((D,), lambda i: (0,)),
            pl.BlockSpec((RB, 1), lambda i: (i, 0)),
            pl.BlockSpec((D, D), lambda i: (0, 0)),
        ],
        out_specs=[
            pl.BlockSpec((RB, DH), lambda i: (i, 0)),
            pl.BlockSpec((RB, DH), lambda i: (i, 0)),
        ],
        out_shape=[
            jax.ShapeDtypeStruct((N, DH), jnp.float32),
            jax.ShapeDtypeStruct((N, DH), jnp.float32),
        ],
    )(s1a, s1b, t_emb, We, be, b1, dinv, W2)


def _final_body(x_ref, s2a_ref, s2b_ref, b2_ref, dinv_ref, out_ref):
    s2 = jnp.concatenate([s2a_ref[...], s2b_ref[...]], axis=1)
    pre = dinv_ref[...] * s2 + b2_ref[...][None, :]
    out_ref[...] = x_ref[...] + pre * jax.nn.sigmoid(pre)


def _final(x, s2a, s2b, b2, dinv):
    return pl.pallas_call(
        _final_body,
        grid=(GRID,),
        in_specs=[
            pl.BlockSpec((RB, D), lambda i: (i, 0)),
            pl.BlockSpec((RB, DH), lambda i: (i, 0)),
            pl.BlockSpec((RB, DH), lambda i: (i, 0)),
            pl.BlockSpec((D,), lambda i: (0,)),
            pl.BlockSpec((RB, 1), lambda i: (i, 0)),
        ],
        out_specs=pl.BlockSpec((RB, D), lambda i: (i, 0)),
        out_shape=jax.ShapeDtypeStruct((N, D), jnp.float32),
    )(x, s2a, s2b, b2, dinv)


def kernel(x, edge_index, t_emb, W1, b1, W2, b2, We, be):
    edges_deg = edge_index.reshape(2, 32, DNCH, DCH)
    edges_sc = edge_index.reshape(2, NSUB, NPH, PCH, CH)
    edges_sc2 = edge_index.reshape(2, NSUB, NPH, PCH, 2, CH // 2)
    ones_d = jnp.ones((DCH,), jnp.float32)
    zeros_n = jnp.zeros((N,), jnp.float32)

    p0, p1 = _deg_kernel(edges_deg, ones_d, zeros_n)
    ga, gb, dinv = _mm1(p0, p1, x, W1)
    s1a, s1b = _scatter_kernel(ga, gb, edges_sc, edges_sc2)
    g2a, g2b = _mm2(s1a, s1b, t_emb, We, be, b1, dinv, W2)
    s2a, s2b = _scatter_kernel(g2a, g2b, edges_sc, edges_sc2)
    return _final(x, s2a, s2b, b2, dinv)


# speedup vs baseline: 1.4492x; 1.0015x over previous
import functools

import jax
import jax.numpy as jnp
from jax import lax
from jax.experimental import pallas as pl
from jax.experimental.pallas import tpu as pltpu
from jax.experimental.pallas import tpu_sc as plsc

N = 10000
E = 160000
D = 256
DH = 128
DT = 512
NSUB = 16
EPT = E // NSUB
CH = 80
NCH = EPT // CH
NPH = 5
PCH = NCH // NPH
SLAB = 624
SLAB_LAST = N - 15 * SLAB
EPW = E // 32
DCH = 40
DNCH = EPW // DCH

_mesh = plsc.VectorSubcoreMesh(core_axis_name="c", subcore_axis_name="s")


@functools.partial(
    pl.kernel,
    mesh=_mesh,
    out_type=(
        jax.ShapeDtypeStruct((N,), jnp.float32),
        jax.ShapeDtypeStruct((N,), jnp.float32),
    ),
    scratch_types=[
        pltpu.VMEM((DNCH, DCH), jnp.int32),
        pltpu.VMEM((DCH,), jnp.float32),
        pltpu.VMEM_SHARED((N,), jnp.float32),
        pltpu.SemaphoreType.DMA,
    ],
)
def _deg_kernel(edges_hbm, ones_hbm, zeros_hbm, out0_hbm, out1_hbm,
                dst_v, ones_v, acc, sem):
    c = lax.axis_index("c")
    s = lax.axis_index("s")
    wid = c * NSUB + s
    pltpu.sync_copy(edges_hbm.at[1, wid], dst_v)
    pltpu.sync_copy(ones_hbm, ones_v)

    @pl.when(s == 0)
    def _():
        pltpu.sync_copy(zeros_hbm, acc)

    plsc.subcore_barrier()

    def body(k, carry):
        pltpu.async_copy(ones_v, acc.at[dst_v.at[k]], sem, add=True)
        return carry

    lax.fori_loop(0, DNCH, body, 0)

    def drain(k, carry):
        pltpu.make_async_copy(ones_v, acc.at[dst_v.at[k]], sem).wait()
        return carry

    lax.fori_loop(0, DNCH, drain, 0)
    plsc.subcore_barrier()

    @pl.when((s == 0) & (c == 0))
    def _():
        pltpu.sync_copy(acc, out0_hbm)

    @pl.when((s == 0) & (c == 1))
    def _():
        pltpu.sync_copy(acc, out1_hbm)


@functools.partial(
    pl.kernel,
    mesh=_mesh,
    out_type=(
        jax.ShapeDtypeStruct((N, DH), jnp.float32),
        jax.ShapeDtypeStruct((N, DH), jnp.float32),
    ),
    scratch_types=[
        pltpu.VMEM((PCH, CH), jnp.int32),
        pltpu.VMEM((PCH, CH), jnp.int32),
        pltpu.VMEM((CH, DH), jnp.float32),
        pltpu.VMEM((CH, DH), jnp.float32),
        pltpu.VMEM((CH, DH), jnp.float32),
        pltpu.VMEM((CH, DH), jnp.float32),
        pltpu.VMEM_SHARED((N, DH), jnp.float32),
        pltpu.SemaphoreType.DMA,
        pltpu.SemaphoreType.DMA,
        pltpu.SemaphoreType.DMA,
        pltpu.SemaphoreType.DMA,
        pltpu.SemaphoreType.DMA,
        pltpu.SemaphoreType.DMA,
        pltpu.SemaphoreType.DMA,
        pltpu.SemaphoreType.DMA,
    ],
)
def _scatter_kernel(ga_hbm, gb_hbm, edges_hbm,
                    outa_hbm, outb_hbm, src_v, dst_v,
                    gbuf_a, gbuf_b, gbuf_c, gbuf_d, acc,
                    sg_a, sg_b, sg_c, sg_d, ss_a, ss_b, ss_c, ss_d):
    c = lax.axis_index("c")
    s = lax.axis_index("s")

    def init_selfloop():
        @pl.when((c == 0) & (s < 15))
        def _():
            pltpu.sync_copy(ga_hbm.at[pl.ds(s * SLAB, SLAB)],
                            acc.at[pl.ds(s * SLAB, SLAB)])

        @pl.when((c == 0) & (s == 15))
        def _():
            pltpu.sync_copy(ga_hbm.at[pl.ds(15 * SLAB, SLAB_LAST)],
                            acc.at[pl.ds(15 * SLAB, SLAB_LAST)])

        @pl.when((c == 1) & (s < 15))
        def _():
            pltpu.sync_copy(gb_hbm.at[pl.ds(s * SLAB, SLAB)],
                            acc.at[pl.ds(s * SLAB, SLAB)])

        @pl.when((c == 1) & (s == 15))
        def _():
            pltpu.sync_copy(gb_hbm.at[pl.ds(15 * SLAB, SLAB_LAST)],
                            acc.at[pl.ds(15 * SLAB, SLAB_LAST)])

    def gstart(k, buf, sem):
        @pl.when(c == 0)
        def _():
            pltpu.async_copy(ga_hbm.at[src_v.at[k]], buf, sem)

        @pl.when(c == 1)
        def _():
            pltpu.async_copy(gb_hbm.at[src_v.at[k]], buf, sem)

    def gwait(k, buf, sem):
        @pl.when(c == 0)
        def _():
            pltpu.make_async_copy(ga_hbm.at[src_v.at[k]], buf, sem).wait()

        @pl.when(c == 1)
        def _():
            pltpu.make_async_copy(gb_hbm.at[src_v.at[k]], buf, sem).wait()

    def sstart(k, buf, sem):
        pltpu.async_copy(buf, acc.at[dst_v.at[k]], sem, add=True)

    def swait(k, buf, sem):
        pltpu.make_async_copy(buf, acc.at[dst_v.at[k]], sem).wait()

    bufs = ((gbuf_a, sg_a, ss_a), (gbuf_b, sg_b, ss_b),
            (gbuf_c, sg_c, ss_c), (gbuf_d, sg_d, ss_d))
    NBUF = 4

    def phase(p, pcarry):
        pltpu.sync_copy(edges_hbm.at[0, s, p], src_v)
        pltpu.sync_copy(edges_hbm.at[1, s, p], dst_v)
        gstart(0, bufs[0][0], bufs[0][1])
        gstart(1, bufs[1][0], bufs[1][1])
        gstart(2, bufs[2][0], bufs[2][1])

        @pl.when(p == 0)
        def _():
            init_selfloop()
            plsc.subcore_barrier()

        def body(k, carry):
            for r in range(NBUF):
                @pl.when(lax.rem(k, NBUF) == r)
                def _(r=r):
                    b = bufs[r]
                    bprev = bufs[(r + NBUF - 1) % NBUF]
                    bnext3 = bufs[(r + 3) % NBUF]
                    gwait(k, b[0], b[1])

                    @pl.when(k >= 1)
                    def _():
                        swait(k - 1, bprev[0], bprev[2])

                    @pl.when(k + 3 < PCH)
                    def _():
                        gstart(k + 3, bnext3[0], bnext3[1])

                    sstart(k, b[0], b[2])

            return carry

        lax.fori_loop(0, PCH, body, 0)
        blast = bufs[(PCH - 1) % NBUF]
        swait(PCH - 1, blast[0], blast[2])
        return pcarry

    lax.fori_loop(0, NPH, phase, 0)
    plsc.subcore_barrier()

    @pl.when((c == 0) & (s < 15))
    def _():
        pltpu.sync_copy(acc.at[pl.ds(s * SLAB, SLAB)],
                        outa_hbm.at[pl.ds(s * SLAB, SLAB)])

    @pl.when((c == 0) & (s == 15))
    def _():
        pltpu.sync_copy(acc.at[pl.ds(15 * SLAB, SLAB_LAST)],
                        outa_hbm.at[pl.ds(15 * SLAB, SLAB_LAST)])

    @pl.when((c == 1) & (s < 15))
    def _():
        pltpu.sync_copy(acc.at[pl.ds(s * SLAB, SLAB)],
                        outb_hbm.at[pl.ds(s * SLAB, SLAB)])

    @pl.when((c == 1) & (s == 15))
    def _():
        pltpu.sync_copy(acc.at[pl.ds(15 * SLAB, SLAB_LAST)],
                        outb_hbm.at[pl.ds(15 * SLAB, SLAB_LAST)])


RB = 1000
GRID = N // RB


def _mm1_body(p0_ref, p1_ref, x_ref, w1_ref, ga_ref, gb_ref, dinv_ref,
              dinv_scr):
    i = pl.program_id(0)

    @pl.when(i == 0)
    def _():
        deg = 1.0 + p0_ref[...] + p1_ref[...]
        dinv_scr[...] = lax.rsqrt(deg)[:, None]

    @pl.when(i > 0)
    def _():
        r = i - 1
        dinv = dinv_scr[pl.ds(r * RB, RB), :]
        h1 = jnp.dot(x_ref[...], w1_ref[...],
                     preferred_element_type=jnp.float32)
        g1 = h1 * dinv
        ga_ref[...] = g1[:, :DH]
        gb_ref[...] = g1[:, DH:]
        dinv_ref[...] = dinv


def _mm1(p0, p1, x, W1):
    return pl.pallas_call(
        _mm1_body,
        grid=(GRID + 1,),
        in_specs=[
            pl.BlockSpec((N,), lambda i: (0,)),
            pl.BlockSpec((N,), lambda i: (0,)),
            pl.BlockSpec((RB, D), lambda i: (jnp.maximum(i - 1, 0), 0)),
            pl.BlockSpec((D, D), lambda i: (0, 0)),
        ],
        out_specs=[
            pl.BlockSpec((RB, DH), lambda i: (jnp.maximum(i - 1, 0), 0)),
            pl.BlockSpec((RB, DH), lambda i: (jnp.maximum(i - 1, 0), 0)),
            pl.BlockSpec((RB, 1), lambda i: (jnp.maximum(i - 1, 0), 0)),
        ],
        out_shape=[
            jax.ShapeDtypeStruct((N, DH), jnp.float32),
            jax.ShapeDtypeStruct((N, DH), jnp.float32),
            jax.ShapeDtypeStruct((N, 1), jnp.float32),
        ],
        scratch_shapes=[pltpu.VMEM((N, 1), jnp.float32)],
    )(p0, p1, x, W1)


def _mm2_body(s1a_ref, s1b_ref, te_ref, we_ref, be_ref,
              b1_ref, dinv_ref, w2_ref, g2a_ref, g2b_ref):
    s1 = jnp.concatenate([s1a_ref[...], s1b_ref[...]], axis=1)
    dinv = dinv_ref[...]
    tt = jnp.dot(te_ref[...], we_ref[...],
                 preferred_element_type=jnp.float32)
    tt = tt + be_ref[...][None, :]
    t = tt * jax.nn.sigmoid(tt)
    h = dinv * s1 + b1_ref[...][None, :] + t
    g2 = jnp.dot(h, w2_ref[...],
                 preferred_element_type=jnp.float32) * dinv
    g2a_ref[...] = g2[:, :DH]
    g2b_ref[...] = g2[:, DH:]


def _mm2(s1a, s1b, t_emb, We, be, b1, dinv, W2):
    return pl.pallas_call(
        _mm2_body,
        grid=(GRID,),
        in_specs=[
            pl.BlockSpec((RB, DH), lambda i: (i, 0)),
            pl.BlockSpec((RB, DH), lambda i: (i, 0)),
            pl.BlockSpec((RB, DT), lambda i: (i, 0)),
            pl.BlockSpec((DT, D), lambda i: (0, 0)),
            pl.BlockSpec((D,), lambda i: (0,)),
            pl.BlockSpec((D,), lambda i: (0,)),
            pl.BlockSpec((RB, 1), lambda i: (i, 0)),
            pl.BlockSpec((D, D), lambda i: (0, 0)),
        ],
        out_specs=[
            pl.BlockSpec((RB, DH), lambda i: (i, 0)),
            pl.BlockSpec((RB, DH), lambda i: (i, 0)),
        ],
        out_shape=[
            jax.ShapeDtypeStruct((N, DH), jnp.float32),
            jax.ShapeDtypeStruct((N, DH), jnp.float32),
        ],
    )(s1a, s1b, t_emb, We, be, b1, dinv, W2)


def _final_body(x_ref, s2a_ref, s2b_ref, b2_ref, dinv_ref, out_ref):
    s2 = jnp.concatenate([s2a_ref[...], s2b_ref[...]], axis=1)
    pre = dinv_ref[...] * s2 + b2_ref[...][None, :]
    out_ref[...] = x_ref[...] + pre * jax.nn.sigmoid(pre)


def _final(x, s2a, s2b, b2, dinv):
    return pl.pallas_call(
        _final_body,
        grid=(GRID,),
        in_specs=[
            pl.BlockSpec((RB, D), lambda i: (i, 0)),
            pl.BlockSpec((RB, DH), lambda i: (i, 0)),
            pl.BlockSpec((RB, DH), lambda i: (i, 0)),
            pl.BlockSpec((D,), lambda i: (0,)),
            pl.BlockSpec((RB, 1), lambda i: (i, 0)),
        ],
        out_specs=pl.BlockSpec((RB, D), lambda i: (i, 0)),
        out_shape=jax.ShapeDtypeStruct((N, D), jnp.float32),
    )(x, s2a, s2b, b2, dinv)


def kernel(x, edge_index, t_emb, W1, b1, W2, b2, We, be):
    edges_deg = edge_index.reshape(2, 32, DNCH, DCH)
    edges_sc = edge_index.reshape(2, NSUB, NPH, PCH, CH)
    ones_d = jnp.ones((DCH,), jnp.float32)
    zeros_n = jnp.zeros((N,), jnp.float32)

    p0, p1 = _deg_kernel(edges_deg, ones_d, zeros_n)
    ga, gb, dinv = _mm1(p0, p1, x, W1)
    s1a, s1b = _scatter_kernel(ga, gb, edges_sc)
    g2a, g2b = _mm2(s1a, s1b, t_emb, We, be, b1, dinv, W2)
    s2a, s2b = _scatter_kernel(g2a, g2b, edges_sc)
    return _final(x, s2a, s2b, b2, dinv)
